# baseline (speedup 1.0000x reference)
import functools

import jax
import jax.numpy as jnp
from jax import lax
from jax.experimental import pallas as pl
from jax.experimental.pallas import tpu as pltpu

D = 256
N = 10000
E = 160000
T2 = 160000
T1 = 80000

ROW_BLK = 1000


def _silu(v):
    return v * jax.nn.sigmoid(v)


def _tc1_body(x_ref, wx1_ref, bx1_ref, wti_ref, wtj_ref, h_ref, ti_ref, tj_ref):
    h = _silu(jnp.dot(x_ref[...], wx1_ref[...],
                      preferred_element_type=jnp.float32) + bx1_ref[...])
    h_ref[...] = h
    ti_ref[...] = jnp.dot(h, wti_ref[...], preferred_element_type=jnp.float32)
    tj_ref[...] = jnp.dot(h, wtj_ref[...], preferred_element_type=jnp.float32)


def _tc1(x, wx1, bx1, wti, wtj):
    nblk = N // ROW_BLK
    full = lambda shp: pl.BlockSpec(shp, lambda i: (0, 0))
    return pl.pallas_call(
        _tc1_body,
        grid=(nblk,),
        in_specs=[
            pl.BlockSpec((ROW_BLK, D), lambda i: (i, 0)),
            full((D, D)), full((1, D)), full((D, 2 * D)), full((D, 2 * D)),
        ],
        out_specs=[
            pl.BlockSpec((ROW_BLK, D), lambda i: (i, 0)),
            pl.BlockSpec((ROW_BLK, 2 * D), lambda i: (i, 0)),
            pl.BlockSpec((ROW_BLK, 2 * D), lambda i: (i, 0)),
        ],
        out_shape=[
            jax.ShapeDtypeStruct((N, D), jnp.float32),
            jax.ShapeDtypeStruct((N, 2 * D), jnp.float32),
            jax.ShapeDtypeStruct((N, 2 * D), jnp.float32),
        ],
    )(x, wx1, bx1, wti, wtj)


def _mlp2_body(x_ref, w1_ref, b1_ref, w2_ref, b2_ref, o0_ref, o1_ref):
    s = _silu(jnp.dot(x_ref[...], w1_ref[...],
                      preferred_element_type=jnp.float32) + b1_ref[...])
    o = _silu(jnp.dot(s, w2_ref[...],
                      preferred_element_type=jnp.float32) + b2_ref[...])
    o0_ref[...] = o[:, :D // 2]
    o1_ref[...] = o[:, D // 2:]


def _sbf_mlp(sbf, w1, b1, w2, b2):
    rows = sbf.shape[0]
    nblk = rows // ROW_BLK
    full = lambda shp: pl.BlockSpec(shp, lambda i: (0, 0))
    return pl.pallas_call(
        _mlp2_body,
        grid=(nblk,),
        in_specs=[
            pl.BlockSpec((ROW_BLK, D), lambda i: (i, 0)),
            full((D, D)), full((1, D)), full((D, D)), full((1, D)),
        ],
        out_specs=[
            pl.BlockSpec((ROW_BLK, D // 2), lambda i: (i, 0)),
            pl.BlockSpec((ROW_BLK, D // 2), lambda i: (i, 0)),
        ],
        out_shape=[
            jax.ShapeDtypeStruct((rows, D // 2), jnp.float32),
            jax.ShapeDtypeStruct((rows, D // 2), jnp.float32),
        ],
    )(sbf, w1, b1, w2, b2)


def _tc24_body(rbf_ref, ga_ref, gb_ref, wr4_ref, bji_ref, bkj_ref,
               nb0_ref, nb1_ref, ro0_ref, ro1_ref, ms0_ref, ms1_ref):
    r4 = jnp.dot(rbf_ref[...], wr4_ref[...], preferred_element_type=jnp.float32)
    ga = ga_ref[...]
    gb = gb_ref[...]
    m_ji = _silu(ga[:, :D] + gb[:, :D] + r4[:, :D] + bji_ref[...])
    m_nb = _silu(ga[:, D:] + gb[:, D:] + r4[:, D:2 * D] + bkj_ref[...]) \
        * r4[:, 2 * D:3 * D]
    rbfo = r4[:, 3 * D:]
    mjis = rbfo * m_ji
    h = D // 2
    nb0_ref[...] = m_nb[:, :h]
    nb1_ref[...] = m_nb[:, h:]
    ro0_ref[...] = rbfo[:, :h]
    ro1_ref[...] = rbfo[:, h:]
    ms0_ref[...] = mjis[:, :h]
    ms1_ref[...] = mjis[:, h:]


def _tc24(rbf, ga, gb, wr4, bji, bkj):
    nblk = E // ROW_BLK
    full = lambda shp: pl.BlockSpec(shp, lambda i: (0, 0))
    half_spec = pl.BlockSpec((ROW_BLK, D // 2), lambda i: (i, 0))
    half_shape = jax.ShapeDtypeStruct((E, D // 2), jnp.float32)
    return pl.pallas_call(
        _tc24_body,
        grid=(nblk,),
        in_specs=[
            pl.BlockSpec((ROW_BLK, D), lambda i: (i, 0)),
            pl.BlockSpec((ROW_BLK, 2 * D), lambda i: (i, 0)),
            pl.BlockSpec((ROW_BLK, 2 * D), lambda i: (i, 0)),
            full((D, 4 * D)), full((1, D)), full((1, D)),
        ],
        out_specs=[half_spec] * 6,
        out_shape=[half_shape] * 6,
    )(rbf, ga, gb, wr4, bji, bkj)


def _tc5_body(x_ref, h_ref, a0_ref, a1_ref,
              wx2_ref, bx2_ref,
              w1a_ref, b1a_ref, w1b_ref, b1b_ref,
              w2a_ref, b2a_ref, w2b_ref, b2b_ref,
              w3a_ref, b3a_ref, w3b_ref, b3b_ref,
              wo1_ref, bo1_ref, wo2_ref, bo2_ref, wo3_ref, bo3_ref,
              wsc_ref, bsc_ref,
              hout_ref, s_ref):
    mm = lambda a, b: jnp.dot(a, b, preferred_element_type=jnp.float32)
    agg = jnp.concatenate([a0_ref[...], a1_ref[...]], axis=1)
    hh = h_ref[...] + agg
    hh = _silu(mm(hh, wx2_ref[...]) + bx2_ref[...])

    def res(zz, wa, ba, wb, bb):
        zo = _silu(mm(_silu(mm(zz, wa[...]) + ba[...]), wb[...]) + bb[...])
        return zo + zz

    hh = res(hh, w1a_ref, b1a_ref, w1b_ref, b1b_ref) + x_ref[...]
    hh = res(hh, w2a_ref, b2a_ref, w2b_ref, b2b_ref)
    hh = res(hh, w3a_ref, b3a_ref, w3b_ref, b3b_ref)
    hout_ref[...] = hh
    out = _silu(mm(hh, wo1_ref[...]) + bo1_ref[...])
    out = _silu(mm(out, wo2_ref[...]) + bo2_ref[...])
    out = _silu(mm(out, wo3_ref[...]) + bo3_ref[...])
    s_ref[...] = mm(out, wsc_ref[...]) + bsc_ref[...]


def _tc5(x, h, a0, a1, weights):
    nblk = N // ROW_BLK
    full = lambda shp: pl.BlockSpec(shp, lambda i: (0, 0))
    row = lambda w: pl.BlockSpec((ROW_BLK, w), lambda i: (i, 0))
    wspecs = []
    for w in weights:
        wspecs.append(full(w.shape))
    return pl.pallas_call(
        _tc5_body,
        grid=(nblk,),
        in_specs=[row(D), row(D), row(D // 2), row(D // 2)] + wspecs,
        out_specs=[row(D), row(D // 2)],
        out_shape=[
            jax.ShapeDtypeStruct((N, D), jnp.float32),
            jax.ShapeDtypeStruct((N, D // 2), jnp.float32),
        ],
    )(x, h, a0, a1, *weights)


def _sc1(ti, tj, i_idx, j_idx):
    return jnp.take(ti, i_idx, axis=0), jnp.take(tj, j_idx, axis=0)


def _sc2(nb0, nb1, ro0, ro1, ms0, ms1, sh20, sh21, sh10, sh11,
         idx_kj, idx_ji, idx_jj_pair, idx_ji_pair, i_idx):
    nb = jnp.concatenate([nb0, nb1], axis=1)
    ro = jnp.concatenate([ro0, ro1], axis=1)
    ms = jnp.concatenate([ms0, ms1], axis=1)
    sh2 = jnp.concatenate([sh20, sh21], axis=1)
    sh1 = jnp.concatenate([sh10, sh11], axis=1)
    dst2 = jnp.take(i_idx, idx_ji, axis=0)
    dst1 = jnp.take(i_idx, idx_ji_pair, axis=0)
    c2 = jnp.take(nb, idx_kj, axis=0) * jnp.take(ro, idx_ji, axis=0) * sh2
    c1 = jnp.take(nb, idx_jj_pair, axis=0) * jnp.take(ro, idx_ji_pair, axis=0) * sh1
    agg = jax.ops.segment_sum(ms, i_idx, num_segments=N)
    agg = agg + jax.ops.segment_sum(c2, dst2, num_segments=N)
    agg = agg + jax.ops.segment_sum(c1, dst1, num_segments=N)
    return agg[:, :D // 2], agg[:, D // 2:]


def kernel(x, rbf, sbf2, sbf1, idx_kj, idx_ji, idx_jj_pair, idx_ji_pair,
           edge_index, params):
    p = params
    j_idx = edge_index[0]
    i_idx = edge_index[1]

    wti = jnp.concatenate([p['Wji'][:D], p['Wkj'][:D]], axis=1)
    wtj = jnp.concatenate([p['Wji'][D:2 * D], p['Wkj'][D:2 * D]], axis=1)
    wr4 = jnp.concatenate(
        [p['Wji'][2 * D:], p['Wkj'][2 * D:], p['Wrbf'], p['Wrbfo']], axis=1)
    wsc = jnp.concatenate(
        [p['Wout'], p['Watt'], jnp.zeros((D, D // 2 - 2), jnp.float32)], axis=1)
    bsc = jnp.concatenate(
        [p['bout'], jnp.zeros((D // 2 - 1,), jnp.float32)])[None]
    row_b = lambda b: b[None]

    h, ti, tj = _tc1(x, p['Wx1'], row_b(p['bx1']), wti, wtj)

    sh20, sh21 = _sbf_mlp(sbf2, p['Wsbf1'], row_b(p['bsbf1']),
                          p['Wsbf2'], row_b(p['bsbf2']))
    sh10, sh11 = _sbf_mlp(sbf1, p['Wsbf1'], row_b(p['bsbf1']),
                          p['Wsbf2'], row_b(p['bsbf2']))

    ga, gb = _sc1(ti, tj, i_idx, j_idx)

    nb0, nb1, ro0, ro1, ms0, ms1 = _tc24(
        rbf, ga, gb, wr4, row_b(p['bji']), row_b(p['bkj']))

    a0, a1 = _sc2(nb0, nb1, ro0, ro1, ms0, ms1, sh20, sh21, sh10, sh11,
                  idx_kj, idx_ji, idx_jj_pair, idx_ji_pair, i_idx)

    tc5_weights = [
        p['Wx2'], row_b(p['bx2']),
        p['Wr1a'], row_b(p['br1a']), p['Wr1b'], row_b(p['br1b']),
        p['Wr2a'], row_b(p['br2a']), p['Wr2b'], row_b(p['br2b']),
        p['Wr3a'], row_b(p['br3a']), p['Wr3b'], row_b(p['br3b']),
        p['Wo1'], row_b(p['bo1']), p['Wo2'], row_b(p['bo2']),
        p['Wo3'], row_b(p['bo3']),
        wsc, bsc,
    ]
    h_out, s = _tc5(x, h, a0, a1, tc5_weights)

    out_final = s[:, 0:1][None]
    att_score = s[:, 1:2][None]
    return (h_out, out_final, att_score)


# speedup vs baseline: 1.2499x; 1.2499x over previous
import functools

import jax
import jax.numpy as jnp
from jax import lax
from jax.experimental import pallas as pl
from jax.experimental.pallas import tpu as pltpu

D = 256
N = 10000
E = 160000
T2 = 160000
T1 = 80000

ROW_BLK = 1000


def _silu(v):
    return v * jax.nn.sigmoid(v)


def _tc1_body(x_ref, wx1_ref, bx1_ref, wti_ref, wtj_ref, h_ref, ti_ref, tj_ref):
    h = _silu(jnp.dot(x_ref[...], wx1_ref[...],
                      preferred_element_type=jnp.float32) + bx1_ref[...])
    h_ref[...] = h
    ti_ref[...] = jnp.dot(h, wti_ref[...], preferred_element_type=jnp.float32)
    tj_ref[...] = jnp.dot(h, wtj_ref[...], preferred_element_type=jnp.float32)


def _tc1(x, wx1, bx1, wti, wtj):
    nblk = N // ROW_BLK
    full = lambda shp: pl.BlockSpec(shp, lambda i: (0, 0))
    return pl.pallas_call(
        _tc1_body,
        grid=(nblk,),
        in_specs=[
            pl.BlockSpec((ROW_BLK, D), lambda i: (i, 0)),
            full((D, D)), full((1, D)), full((D, 2 * D)), full((D, 2 * D)),
        ],
        out_specs=[
            pl.BlockSpec((ROW_BLK, D), lambda i: (i, 0)),
            pl.BlockSpec((ROW_BLK, 2 * D), lambda i: (i, 0)),
            pl.BlockSpec((ROW_BLK, 2 * D), lambda i: (i, 0)),
        ],
        out_shape=[
            jax.ShapeDtypeStruct((N, D), jnp.float32),
            jax.ShapeDtypeStruct((N, 2 * D), jnp.float32),
            jax.ShapeDtypeStruct((N, 2 * D), jnp.float32),
        ],
    )(x, wx1, bx1, wti, wtj)


def _mlp2_body(x_ref, w1_ref, b1_ref, w2_ref, b2_ref, o0_ref, o1_ref):
    s = _silu(jnp.dot(x_ref[...], w1_ref[...],
                      preferred_element_type=jnp.float32) + b1_ref[...])
    o = _silu(jnp.dot(s, w2_ref[...],
                      preferred_element_type=jnp.float32) + b2_ref[...])
    o0_ref[...] = o[:, :D // 2]
    o1_ref[...] = o[:, D // 2:]


def _sbf_mlp(sbf, w1, b1, w2, b2):
    rows = sbf.shape[0]
    nblk = rows // ROW_BLK
    full = lambda shp: pl.BlockSpec(shp, lambda i: (0, 0))
    return pl.pallas_call(
        _mlp2_body,
        grid=(nblk,),
        in_specs=[
            pl.BlockSpec((ROW_BLK, D), lambda i: (i, 0)),
            full((D, D)), full((1, D)), full((D, D)), full((1, D)),
        ],
        out_specs=[
            pl.BlockSpec((ROW_BLK, D // 2), lambda i: (i, 0)),
            pl.BlockSpec((ROW_BLK, D // 2), lambda i: (i, 0)),
        ],
        out_shape=[
            jax.ShapeDtypeStruct((rows, D // 2), jnp.float32),
            jax.ShapeDtypeStruct((rows, D // 2), jnp.float32),
        ],
    )(sbf, w1, b1, w2, b2)


def _tc24_body(rbf_ref, ga_ref, gb_ref, wr4_ref, bji_ref, bkj_ref,
               nb0_ref, nb1_ref, ro0_ref, ro1_ref, ms0_ref, ms1_ref):
    r4 = jnp.dot(rbf_ref[...], wr4_ref[...], preferred_element_type=jnp.float32)
    ga = ga_ref[...]
    gb = gb_ref[...]
    m_ji = _silu(ga[:, :D] + gb[:, :D] + r4[:, :D] + bji_ref[...])
    m_nb = _silu(ga[:, D:] + gb[:, D:] + r4[:, D:2 * D] + bkj_ref[...]) \
        * r4[:, 2 * D:3 * D]
    rbfo = r4[:, 3 * D:]
    mjis = rbfo * m_ji
    h = D // 2
    nb0_ref[...] = m_nb[:, :h]
    nb1_ref[...] = m_nb[:, h:]
    ro0_ref[...] = rbfo[:, :h]
    ro1_ref[...] = rbfo[:, h:]
    ms0_ref[...] = mjis[:, :h]
    ms1_ref[...] = mjis[:, h:]


def _tc24(rbf, ga, gb, wr4, bji, bkj):
    nblk = E // ROW_BLK
    full = lambda shp: pl.BlockSpec(shp, lambda i: (0, 0))
    half_spec = pl.BlockSpec((ROW_BLK, D // 2), lambda i: (i, 0))
    half_shape = jax.ShapeDtypeStruct((E, D // 2), jnp.float32)
    return pl.pallas_call(
        _tc24_body,
        grid=(nblk,),
        in_specs=[
            pl.BlockSpec((ROW_BLK, D), lambda i: (i, 0)),
            pl.BlockSpec((ROW_BLK, 2 * D), lambda i: (i, 0)),
            pl.BlockSpec((ROW_BLK, 2 * D), lambda i: (i, 0)),
            full((D, 4 * D)), full((1, D)), full((1, D)),
        ],
        out_specs=[half_spec] * 6,
        out_shape=[half_shape] * 6,
    )(rbf, ga, gb, wr4, bji, bkj)


def _tc5_body(x_ref, h_ref, a0_ref, a1_ref,
              wx2_ref, bx2_ref,
              w1a_ref, b1a_ref, w1b_ref, b1b_ref,
              w2a_ref, b2a_ref, w2b_ref, b2b_ref,
              w3a_ref, b3a_ref, w3b_ref, b3b_ref,
              wo1_ref, bo1_ref, wo2_ref, bo2_ref, wo3_ref, bo3_ref,
              wsc_ref, bsc_ref,
              hout_ref, s_ref):
    mm = lambda a, b: jnp.dot(a, b, preferred_element_type=jnp.float32)
    agg = jnp.concatenate([a0_ref[...], a1_ref[...]], axis=1)
    hh = h_ref[...] + agg
    hh = _silu(mm(hh, wx2_ref[...]) + bx2_ref[...])

    def res(zz, wa, ba, wb, bb):
        zo = _silu(mm(_silu(mm(zz, wa[...]) + ba[...]), wb[...]) + bb[...])
        return zo + zz

    hh = res(hh, w1a_ref, b1a_ref, w1b_ref, b1b_ref) + x_ref[...]
    hh = res(hh, w2a_ref, b2a_ref, w2b_ref, b2b_ref)
    hh = res(hh, w3a_ref, b3a_ref, w3b_ref, b3b_ref)
    hout_ref[...] = hh
    out = _silu(mm(hh, wo1_ref[...]) + bo1_ref[...])
    out = _silu(mm(out, wo2_ref[...]) + bo2_ref[...])
    out = _silu(mm(out, wo3_ref[...]) + bo3_ref[...])
    s_ref[...] = mm(out, wsc_ref[...]) + bsc_ref[...]


def _tc5(x, h, a0, a1, weights):
    nblk = N // ROW_BLK
    full = lambda shp: pl.BlockSpec(shp, lambda i: (0, 0))
    row = lambda w: pl.BlockSpec((ROW_BLK, w), lambda i: (i, 0))
    wspecs = []
    for w in weights:
        wspecs.append(full(w.shape))
    return pl.pallas_call(
        _tc5_body,
        grid=(nblk,),
        in_specs=[row(D), row(D), row(D // 2), row(D // 2)] + wspecs,
        out_specs=[row(D), row(D // 2)],
        out_shape=[
            jax.ShapeDtypeStruct((N, D), jnp.float32),
            jax.ShapeDtypeStruct((N, D // 2), jnp.float32),
        ],
    )(x, h, a0, a1, *weights)


from jax.experimental.pallas import tpu_sc as plsc

_NW = 32
_EPW = E // _NW
_K1 = 120
_NCH1 = _EPW // _K1
_TAIL1 = _EPW - _NCH1 * _K1


def _sc1_body(ti_h, tj_h, i_h, j_h, ga_h, gb_h,
              ii_v, jj_v, ba_v, bb_v, sa, sb):
    c = lax.axis_index("c")
    s = lax.axis_index("s")
    w = s * 2 + c
    base = w * _EPW

    def do_chunk(off, n):
        ii = ii_v if n == _K1 else ii_v.at[pl.ds(0, n)]
        jj = jj_v if n == _K1 else jj_v.at[pl.ds(0, n)]
        ba = ba_v if n == _K1 else ba_v.at[pl.ds(0, n)]
        bb = bb_v if n == _K1 else bb_v.at[pl.ds(0, n)]
        pltpu.sync_copy(i_h.at[pl.ds(off, n)], ii)
        pltpu.sync_copy(j_h.at[pl.ds(off, n)], jj)
        cpa = pltpu.async_copy(ti_h.at[ii], ba, sa)
        cpb = pltpu.async_copy(tj_h.at[jj], bb, sb)
        cpa.wait()
        cpb.wait()
        pltpu.sync_copy(ba, ga_h.at[pl.ds(off, n)])
        pltpu.sync_copy(bb, gb_h.at[pl.ds(off, n)])

    def step(ci, carry):
        do_chunk(base + ci * _K1, _K1)
        return carry

    lax.fori_loop(0, _NCH1, step, 0)
    do_chunk(base + _NCH1 * _K1, _TAIL1)


def _sc1(ti, tj, edge_index):
    mesh = plsc.VectorSubcoreMesh(core_axis_name="c", subcore_axis_name="s")
    f = functools.partial(
        pl.kernel,
        out_type=[
            jax.ShapeDtypeStruct((E, 2 * D), jnp.float32),
            jax.ShapeDtypeStruct((E, 2 * D), jnp.float32),
        ],
        mesh=mesh,
        scratch_types=[
            pltpu.VMEM((_K1,), jnp.int32),
            pltpu.VMEM((_K1,), jnp.int32),
            pltpu.VMEM((_K1, 2 * D), jnp.float32),
            pltpu.VMEM((_K1, 2 * D), jnp.float32),
            pltpu.SemaphoreType.DMA,
            pltpu.SemaphoreType.DMA,
        ],
    )(_sc1_body)
    return f(ti, tj, edge_index[1], edge_index[0])


def _sc2(nb0, nb1, ro0, ro1, ms0, ms1, sh20, sh21, sh10, sh11,
         idx_kj, idx_ji, idx_jj_pair, idx_ji_pair, i_idx):
    nb = jnp.concatenate([nb0, nb1], axis=1)
    ro = jnp.concatenate([ro0, ro1], axis=1)
    ms = jnp.concatenate([ms0, ms1], axis=1)
    sh2 = jnp.concatenate([sh20, sh21], axis=1)
    sh1 = jnp.concatenate([sh10, sh11], axis=1)
    dst2 = jnp.take(i_idx, idx_ji, axis=0)
    dst1 = jnp.take(i_idx, idx_ji_pair, axis=0)
    c2 = jnp.take(nb, idx_kj, axis=0) * jnp.take(ro, idx_ji, axis=0) * sh2
    c1 = jnp.take(nb, idx_jj_pair, axis=0) * jnp.take(ro, idx_ji_pair, axis=0) * sh1
    agg = jax.ops.segment_sum(ms, i_idx, num_segments=N)
    agg = agg + jax.ops.segment_sum(c2, dst2, num_segments=N)
    agg = agg + jax.ops.segment_sum(c1, dst1, num_segments=N)
    return agg[:, :D // 2], agg[:, D // 2:]


def kernel(x, rbf, sbf2, sbf1, idx_kj, idx_ji, idx_jj_pair, idx_ji_pair,
           edge_index, params):
    p = params
    j_idx = edge_index[0]
    i_idx = edge_index[1]

    wti = jnp.concatenate([p['Wji'][:D], p['Wkj'][:D]], axis=1)
    wtj = jnp.concatenate([p['Wji'][D:2 * D], p['Wkj'][D:2 * D]], axis=1)
    wr4 = jnp.concatenate(
        [p['Wji'][2 * D:], p['Wkj'][2 * D:], p['Wrbf'], p['Wrbfo']], axis=1)
    wsc = jnp.concatenate(
        [p['Wout'], p['Watt'], jnp.zeros((D, D // 2 - 2), jnp.float32)], axis=1)
    bsc = jnp.concatenate(
        [p['bout'], jnp.zeros((D // 2 - 1,), jnp.float32)])[None]
    row_b = lambda b: b[None]

    h, ti, tj = _tc1(x, p['Wx1'], row_b(p['bx1']), wti, wtj)

    sh20, sh21 = _sbf_mlp(sbf2, p['Wsbf1'], row_b(p['bsbf1']),
                          p['Wsbf2'], row_b(p['bsbf2']))
    sh10, sh11 = _sbf_mlp(sbf1, p['Wsbf1'], row_b(p['bsbf1']),
                          p['Wsbf2'], row_b(p['bsbf2']))

    ga, gb = _sc1(ti, tj, edge_index)

    nb0, nb1, ro0, ro1, ms0, ms1 = _tc24(
        rbf, ga, gb, wr4, row_b(p['bji']), row_b(p['bkj']))

    a0, a1 = _sc2(nb0, nb1, ro0, ro1, ms0, ms1, sh20, sh21, sh10, sh11,
                  idx_kj, idx_ji, idx_jj_pair, idx_ji_pair, i_idx)

    tc5_weights = [
        p['Wx2'], row_b(p['bx2']),
        p['Wr1a'], row_b(p['br1a']), p['Wr1b'], row_b(p['br1b']),
        p['Wr2a'], row_b(p['br2a']), p['Wr2b'], row_b(p['br2b']),
        p['Wr3a'], row_b(p['br3a']), p['Wr3b'], row_b(p['br3b']),
        p['Wo1'], row_b(p['bo1']), p['Wo2'], row_b(p['bo2']),
        p['Wo3'], row_b(p['bo3']),
        wsc, bsc,
    ]
    h_out, s = _tc5(x, h, a0, a1, tc5_weights)

    out_final = s[:, 0:1][None]
    att_score = s[:, 1:2][None]
    return (h_out, out_final, att_score)


# speedup vs baseline: 1.4898x; 1.1920x over previous
import functools

import jax
import jax.numpy as jnp
from jax import lax
from jax.experimental import pallas as pl
from jax.experimental.pallas import tpu as pltpu

D = 256
N = 10000
E = 160000
T2 = 160000
T1 = 80000

ROW_BLK = 1000


def _silu(v):
    return v * jax.nn.sigmoid(v)


def _tc1_body(x_ref, wx1_ref, bx1_ref, wti_ref, wtj_ref, h_ref, ti_ref, tj_ref):
    h = _silu(jnp.dot(x_ref[...], wx1_ref[...],
                      preferred_element_type=jnp.float32) + bx1_ref[...])
    h_ref[...] = h
    ti_ref[...] = jnp.dot(h, wti_ref[...], preferred_element_type=jnp.float32)
    tj_ref[...] = jnp.dot(h, wtj_ref[...], preferred_element_type=jnp.float32)


def _tc1(x, wx1, bx1, wti, wtj):
    nblk = N // ROW_BLK
    full = lambda shp: pl.BlockSpec(shp, lambda i: (0, 0))
    return pl.pallas_call(
        _tc1_body,
        grid=(nblk,),
        in_specs=[
            pl.BlockSpec((ROW_BLK, D), lambda i: (i, 0)),
            full((D, D)), full((1, D)), full((D, 2 * D)), full((D, 2 * D)),
        ],
        out_specs=[
            pl.BlockSpec((ROW_BLK, D), lambda i: (i, 0)),
            pl.BlockSpec((ROW_BLK, 2 * D), lambda i: (i, 0)),
            pl.BlockSpec((ROW_BLK, 2 * D), lambda i: (i, 0)),
        ],
        out_shape=[
            jax.ShapeDtypeStruct((N, D), jnp.float32),
            jax.ShapeDtypeStruct((N, 2 * D), jnp.float32),
            jax.ShapeDtypeStruct((N, 2 * D), jnp.float32),
        ],
    )(x, wx1, bx1, wti, wtj)


def _mlp2_body(x_ref, w1_ref, b1_ref, w2_ref, b2_ref, o0_ref, o1_ref):
    s = _silu(jnp.dot(x_ref[...], w1_ref[...],
                      preferred_element_type=jnp.float32) + b1_ref[...])
    o = _silu(jnp.dot(s, w2_ref[...],
                      preferred_element_type=jnp.float32) + b2_ref[...])
    o0_ref[...] = o[:, :D // 2]
    o1_ref[...] = o[:, D // 2:]


def _sbf_mlp(sbf, w1, b1, w2, b2):
    rows = sbf.shape[0]
    nblk = rows // ROW_BLK
    full = lambda shp: pl.BlockSpec(shp, lambda i: (0, 0))
    return pl.pallas_call(
        _mlp2_body,
        grid=(nblk,),
        in_specs=[
            pl.BlockSpec((ROW_BLK, D), lambda i: (i, 0)),
            full((D, D)), full((1, D)), full((D, D)), full((1, D)),
        ],
        out_specs=[
            pl.BlockSpec((ROW_BLK, D // 2), lambda i: (i, 0)),
            pl.BlockSpec((ROW_BLK, D // 2), lambda i: (i, 0)),
        ],
        out_shape=[
            jax.ShapeDtypeStruct((rows, D // 2), jnp.float32),
            jax.ShapeDtypeStruct((rows, D // 2), jnp.float32),
        ],
    )(sbf, w1, b1, w2, b2)


def _tc24_body(rbf_ref, ga_ref, gb_ref, wr4_ref, bji_ref, bkj_ref,
               nb0_ref, nb1_ref, ro0_ref, ro1_ref, ms0_ref, ms1_ref):
    r4 = jnp.dot(rbf_ref[...], wr4_ref[...], preferred_element_type=jnp.float32)
    ga = ga_ref[...]
    gb = gb_ref[...]
    m_ji = _silu(ga[:, :D] + gb[:, :D] + r4[:, :D] + bji_ref[...])
    m_nb = _silu(ga[:, D:] + gb[:, D:] + r4[:, D:2 * D] + bkj_ref[...]) \
        * r4[:, 2 * D:3 * D]
    rbfo = r4[:, 3 * D:]
    mjis = rbfo * m_ji
    h = D // 2
    nb0_ref[...] = m_nb[:, :h]
    nb1_ref[...] = m_nb[:, h:]
    ro0_ref[...] = rbfo[:, :h]
    ro1_ref[...] = rbfo[:, h:]
    ms0_ref[...] = mjis[:, :h]
    ms1_ref[...] = mjis[:, h:]


def _tc24(rbf, ga, gb, wr4, bji, bkj):
    nblk = E // ROW_BLK
    full = lambda shp: pl.BlockSpec(shp, lambda i: (0, 0))
    half_spec = pl.BlockSpec((ROW_BLK, D // 2), lambda i: (i, 0))
    half_shape = jax.ShapeDtypeStruct((E, D // 2), jnp.float32)
    return pl.pallas_call(
        _tc24_body,
        grid=(nblk,),
        in_specs=[
            pl.BlockSpec((ROW_BLK, D), lambda i: (i, 0)),
            pl.BlockSpec((ROW_BLK, 2 * D), lambda i: (i, 0)),
            pl.BlockSpec((ROW_BLK, 2 * D), lambda i: (i, 0)),
            full((D, 4 * D)), full((1, D)), full((1, D)),
        ],
        out_specs=[half_spec] * 6,
        out_shape=[half_shape] * 6,
    )(rbf, ga, gb, wr4, bji, bkj)


def _tc5_body(x_ref, h_ref, a00_ref, a01_ref, a10_ref, a11_ref,
              wx2_ref, bx2_ref,
              w1a_ref, b1a_ref, w1b_ref, b1b_ref,
              w2a_ref, b2a_ref, w2b_ref, b2b_ref,
              w3a_ref, b3a_ref, w3b_ref, b3b_ref,
              wo1_ref, bo1_ref, wo2_ref, bo2_ref, wo3_ref, bo3_ref,
              wsc_ref, bsc_ref,
              hout_ref, s_ref):
    mm = lambda a, b: jnp.dot(a, b, preferred_element_type=jnp.float32)
    agg = jnp.concatenate([a00_ref[...] + a01_ref[...],
                           a10_ref[...] + a11_ref[...]], axis=1)
    hh = h_ref[...] + agg
    hh = _silu(mm(hh, wx2_ref[...]) + bx2_ref[...])

    def res(zz, wa, ba, wb, bb):
        zo = _silu(mm(_silu(mm(zz, wa[...]) + ba[...]), wb[...]) + bb[...])
        return zo + zz

    hh = res(hh, w1a_ref, b1a_ref, w1b_ref, b1b_ref) + x_ref[...]
    hh = res(hh, w2a_ref, b2a_ref, w2b_ref, b2b_ref)
    hh = res(hh, w3a_ref, b3a_ref, w3b_ref, b3b_ref)
    hout_ref[...] = hh
    out = _silu(mm(hh, wo1_ref[...]) + bo1_ref[...])
    out = _silu(mm(out, wo2_ref[...]) + bo2_ref[...])
    out = _silu(mm(out, wo3_ref[...]) + bo3_ref[...])
    s_ref[...] = mm(out, wsc_ref[...]) + bsc_ref[...]


def _tc5(x, h, a00, a01, a10, a11, weights):
    nblk = N // ROW_BLK
    full = lambda shp: pl.BlockSpec(shp, lambda i: (0, 0))
    row = lambda w: pl.BlockSpec((ROW_BLK, w), lambda i: (i, 0))
    wspecs = []
    for w in weights:
        wspecs.append(full(w.shape))
    return pl.pallas_call(
        _tc5_body,
        grid=(nblk,),
        in_specs=[row(D), row(D)] + [row(D // 2)] * 4 + wspecs,
        out_specs=[row(D), row(D // 2)],
        out_shape=[
            jax.ShapeDtypeStruct((N, D), jnp.float32),
            jax.ShapeDtypeStruct((N, D // 2), jnp.float32),
        ],
    )(x, h, a00, a01, a10, a11, *weights)


from jax.experimental.pallas import tpu_sc as plsc

_NW = 32
_EPW = E // _NW
_K1 = 120
_NCH1 = _EPW // _K1
_TAIL1 = _EPW - _NCH1 * _K1


def _sc1_body(ti_h, tj_h, i_h, j_h, ga_h, gb_h,
              ii_v, jj_v, ba_v, bb_v, sa, sb):
    c = lax.axis_index("c")
    s = lax.axis_index("s")
    w = s * 2 + c
    base = w * _EPW

    def do_chunk(off, n):
        ii = ii_v if n == _K1 else ii_v.at[pl.ds(0, n)]
        jj = jj_v if n == _K1 else jj_v.at[pl.ds(0, n)]
        ba = ba_v if n == _K1 else ba_v.at[pl.ds(0, n)]
        bb = bb_v if n == _K1 else bb_v.at[pl.ds(0, n)]
        pltpu.sync_copy(i_h.at[pl.ds(off, n)], ii)
        pltpu.sync_copy(j_h.at[pl.ds(off, n)], jj)
        cpa = pltpu.async_copy(ti_h.at[ii], ba, sa)
        cpb = pltpu.async_copy(tj_h.at[jj], bb, sb)
        cpa.wait()
        cpb.wait()
        pltpu.sync_copy(ba, ga_h.at[pl.ds(off, n)])
        pltpu.sync_copy(bb, gb_h.at[pl.ds(off, n)])

    def step(ci, carry):
        do_chunk(base + ci * _K1, _K1)
        return carry

    lax.fori_loop(0, _NCH1, step, 0)
    do_chunk(base + _NCH1 * _K1, _TAIL1)


def _sc1(ti, tj, edge_index):
    mesh = plsc.VectorSubcoreMesh(core_axis_name="c", subcore_axis_name="s")
    f = functools.partial(
        pl.kernel,
        out_type=[
            jax.ShapeDtypeStruct((E, 2 * D), jnp.float32),
            jax.ShapeDtypeStruct((E, 2 * D), jnp.float32),
        ],
        mesh=mesh,
        scratch_types=[
            pltpu.VMEM((_K1,), jnp.int32),
            pltpu.VMEM((_K1,), jnp.int32),
            pltpu.VMEM((_K1, 2 * D), jnp.float32),
            pltpu.VMEM((_K1, 2 * D), jnp.float32),
            pltpu.SemaphoreType.DMA,
            pltpu.SemaphoreType.DMA,
        ],
    )(_sc1_body)
    return f(ti, tj, edge_index[1], edge_index[0])


_K2 = 40
_HD = D // 2
_NP = 10240
_RPS = _NP // 16


def _sc2_body(nb_h, ro_h, ms_h, sh2_h, sh1_h,
              ikj_h, iji_h, d2_h, ijj_h, ijp_h, d1_h, ie_h,
              out_h,
              g1_v, g2_v, g3_v, i1_v, i2_v, dv_v, acc_sh,
              s1, s2, s3):
    c = lax.axis_index("c")
    s = lax.axis_index("s")
    w = s * 2 + c

    def zrow(r, carry):
        for cc in range(_HD // 16):
            g1_v[r, pl.ds(cc * 16, 16)] = jnp.zeros((16,), jnp.float32)
        return carry
    lax.fori_loop(0, _K2, zrow, 0)

    def zcopy(k, carry):
        pltpu.sync_copy(g1_v, acc_sh.at[pl.ds(s * _RPS + k * _K2, _K2)])
        return carry
    lax.fori_loop(0, _RPS // _K2, zcopy, 0)
    plsc.subcore_barrier()

    def triplet_stream(idxa_h, idxb_h, dst_h, sh_h, nrows):
        nchunks = nrows // _K2
        niter = (nchunks + _NW - 1) // _NW

        def step(g, carry):
            cid = g * _NW + w

            @pl.when(cid < nchunks)
            def _():
                off = cid * _K2
                pltpu.sync_copy(idxa_h.at[pl.ds(off, _K2)], i1_v)
                pltpu.sync_copy(idxb_h.at[pl.ds(off, _K2)], i2_v)
                pltpu.sync_copy(dst_h.at[pl.ds(off, _K2)], dv_v)
                ca = pltpu.async_copy(nb_h.at[i1_v], g1_v, s1)
                cb = pltpu.async_copy(ro_h.at[i2_v], g2_v, s2)
                cc_ = pltpu.async_copy(sh_h.at[pl.ds(off, _K2)], g3_v, s3)
                ca.wait()
                cb.wait()
                cc_.wait()

                def mrow(r, carry2):
                    for cc in range(_HD // 16):
                        sl = pl.ds(cc * 16, 16)
                        g1_v[r, sl] = g1_v[r, sl] * g2_v[r, sl] * g3_v[r, sl]
                    return carry2
                lax.fori_loop(0, _K2, mrow, 0)
                pltpu.sync_copy(g1_v, acc_sh.at[dv_v], add=True)
            return carry
        lax.fori_loop(0, niter, step, 0)

    triplet_stream(ikj_h, iji_h, d2_h, sh2_h, T2)
    triplet_stream(ijj_h, ijp_h, d1_h, sh1_h, T1)

    nchunks_e = E // _K2

    def estep(g, carry):
        cid = g * _NW + w
        off = cid * _K2
        pltpu.sync_copy(ie_h.at[pl.ds(off, _K2)], dv_v)
        ce = pltpu.async_copy(ms_h.at[pl.ds(off, _K2)], g3_v, s3)
        ce.wait()
        pltpu.sync_copy(g3_v, acc_sh.at[dv_v], add=True)
        return carry
    lax.fori_loop(0, nchunks_e // _NW, estep, 0)

    plsc.subcore_barrier()
    pltpu.sync_copy(acc_sh.at[pl.ds(s * _RPS, _RPS)],
                    out_h.at[c, pl.ds(s * _RPS, _RPS)])


def _sc2_half(nb, ro, ms, sh2, sh1, idx_kj, idx_ji, dst2,
              idx_jj, idx_jp, dst1, i_idx):
    mesh = plsc.VectorSubcoreMesh(core_axis_name="c", subcore_axis_name="s")
    f = functools.partial(
        pl.kernel,
        out_type=jax.ShapeDtypeStruct((2, _NP, _HD), jnp.float32),
        mesh=mesh,
        scratch_types=[
            pltpu.VMEM((_K2, _HD), jnp.float32),
            pltpu.VMEM((_K2, _HD), jnp.float32),
            pltpu.VMEM((_K2, _HD), jnp.float32),
            pltpu.VMEM((_K2,), jnp.int32),
            pltpu.VMEM((_K2,), jnp.int32),
            pltpu.VMEM((_K2,), jnp.int32),
            pltpu.VMEM_SHARED((_NP, _HD), jnp.float32),
            pltpu.SemaphoreType.DMA,
            pltpu.SemaphoreType.DMA,
            pltpu.SemaphoreType.DMA,
        ],
    )(_sc2_body)
    return f(nb, ro, ms, sh2, sh1, idx_kj, idx_ji, dst2,
             idx_jj, idx_jp, dst1, i_idx)


def _sc2(nb0, nb1, ro0, ro1, ms0, ms1, sh20, sh21, sh10, sh11,
         idx_kj, idx_ji, idx_jj_pair, idx_ji_pair, i_idx):
    dst2 = jnp.take(i_idx, idx_ji, axis=0)
    dst1 = jnp.take(i_idx, idx_ji_pair, axis=0)
    p0 = _sc2_half(nb0, ro0, ms0, sh20, sh10, idx_kj, idx_ji, dst2,
                   idx_jj_pair, idx_ji_pair, dst1, i_idx)
    p1 = _sc2_half(nb1, ro1, ms1, sh21, sh11, idx_kj, idx_ji, dst2,
                   idx_jj_pair, idx_ji_pair, dst1, i_idx)
    return p0[0, :N], p0[1, :N], p1[0, :N], p1[1, :N]


def kernel(x, rbf, sbf2, sbf1, idx_kj, idx_ji, idx_jj_pair, idx_ji_pair,
           edge_index, params):
    p = params
    j_idx = edge_index[0]
    i_idx = edge_index[1]

    wti = jnp.concatenate([p['Wji'][:D], p['Wkj'][:D]], axis=1)
    wtj = jnp.concatenate([p['Wji'][D:2 * D], p['Wkj'][D:2 * D]], axis=1)
    wr4 = jnp.concatenate(
        [p['Wji'][2 * D:], p['Wkj'][2 * D:], p['Wrbf'], p['Wrbfo']], axis=1)
    wsc = jnp.concatenate(
        [p['Wout'], p['Watt'], jnp.zeros((D, D // 2 - 2), jnp.float32)], axis=1)
    bsc = jnp.concatenate(
        [p['bout'], jnp.zeros((D // 2 - 1,), jnp.float32)])[None]
    row_b = lambda b: b[None]

    h, ti, tj = _tc1(x, p['Wx1'], row_b(p['bx1']), wti, wtj)

    sh20, sh21 = _sbf_mlp(sbf2, p['Wsbf1'], row_b(p['bsbf1']),
                          p['Wsbf2'], row_b(p['bsbf2']))
    sh10, sh11 = _sbf_mlp(sbf1, p['Wsbf1'], row_b(p['bsbf1']),
                          p['Wsbf2'], row_b(p['bsbf2']))

    ga, gb = _sc1(ti, tj, edge_index)

    nb0, nb1, ro0, ro1, ms0, ms1 = _tc24(
        rbf, ga, gb, wr4, row_b(p['bji']), row_b(p['bkj']))

    a00, a01, a10, a11 = _sc2(nb0, nb1, ro0, ro1, ms0, ms1,
                              sh20, sh21, sh10, sh11,
                              idx_kj, idx_ji, idx_jj_pair, idx_ji_pair, i_idx)

    tc5_weights = [
        p['Wx2'], row_b(p['bx2']),
        p['Wr1a'], row_b(p['br1a']), p['Wr1b'], row_b(p['br1b']),
        p['Wr2a'], row_b(p['br2a']), p['Wr2b'], row_b(p['br2b']),
        p['Wr3a'], row_b(p['br3a']), p['Wr3b'], row_b(p['br3b']),
        p['Wo1'], row_b(p['bo1']), p['Wo2'], row_b(p['bo2']),
        p['Wo3'], row_b(p['bo3']),
        wsc, bsc,
    ]
    h_out, s = _tc5(x, h, a00, a01, a10, a11, tc5_weights)

    out_final = s[:, 0:1][None]
    att_score = s[:, 1:2][None]
    return (h_out, out_final, att_score)


# speedup vs baseline: 2.0610x; 1.3834x over previous
import functools

import jax
import jax.numpy as jnp
from jax import lax
from jax.experimental import pallas as pl
from jax.experimental.pallas import tpu as pltpu

D = 256
N = 10000
E = 160000
T2 = 160000
T1 = 80000

ROW_BLK = 1000


def _silu(v):
    return v * jax.nn.sigmoid(v)


def _tc1_body(x_ref, wx1_ref, bx1_ref, wti_ref, wtj_ref, h_ref, ti_ref, tj_ref):
    h = _silu(jnp.dot(x_ref[...], wx1_ref[...],
                      preferred_element_type=jnp.float32) + bx1_ref[...])
    h_ref[...] = h
    ti_ref[...] = jnp.dot(h, wti_ref[...], preferred_element_type=jnp.float32)
    tj_ref[...] = jnp.dot(h, wtj_ref[...], preferred_element_type=jnp.float32)


def _tc1(x, wx1, bx1, wti, wtj):
    nblk = N // ROW_BLK
    full = lambda shp: pl.BlockSpec(shp, lambda i: (0, 0))
    return pl.pallas_call(
        _tc1_body,
        grid=(nblk,),
        in_specs=[
            pl.BlockSpec((ROW_BLK, D), lambda i: (i, 0)),
            full((D, D)), full((1, D)), full((D, 2 * D)), full((D, 2 * D)),
        ],
        out_specs=[
            pl.BlockSpec((ROW_BLK, D), lambda i: (i, 0)),
            pl.BlockSpec((ROW_BLK, 2 * D), lambda i: (i, 0)),
            pl.BlockSpec((ROW_BLK, 2 * D), lambda i: (i, 0)),
        ],
        out_shape=[
            jax.ShapeDtypeStruct((N, D), jnp.float32),
            jax.ShapeDtypeStruct((N, 2 * D), jnp.float32),
            jax.ShapeDtypeStruct((N, 2 * D), jnp.float32),
        ],
    )(x, wx1, bx1, wti, wtj)


def _mlp2_body(x_ref, w1_ref, b1_ref, w2_ref, b2_ref, o0_ref, o1_ref):
    s = _silu(jnp.dot(x_ref[...], w1_ref[...],
                      preferred_element_type=jnp.float32) + b1_ref[...])
    o = _silu(jnp.dot(s, w2_ref[...],
                      preferred_element_type=jnp.float32) + b2_ref[...])
    o0_ref[...] = o[:, :D // 2]
    o1_ref[...] = o[:, D // 2:]


def _sbf_mlp(sbf, w1, b1, w2, b2):
    rows = sbf.shape[0]
    nblk = rows // ROW_BLK
    full = lambda shp: pl.BlockSpec(shp, lambda i: (0, 0))
    return pl.pallas_call(
        _mlp2_body,
        grid=(nblk,),
        in_specs=[
            pl.BlockSpec((ROW_BLK, D), lambda i: (i, 0)),
            full((D, D)), full((1, D)), full((D, D)), full((1, D)),
        ],
        out_specs=[
            pl.BlockSpec((ROW_BLK, D // 2), lambda i: (i, 0)),
            pl.BlockSpec((ROW_BLK, D // 2), lambda i: (i, 0)),
        ],
        out_shape=[
            jax.ShapeDtypeStruct((rows, D // 2), jnp.float32),
            jax.ShapeDtypeStruct((rows, D // 2), jnp.float32),
        ],
    )(sbf, w1, b1, w2, b2)


def _tc24_body(rbf_ref, ga_ref, gb_ref, wr4_ref, bji_ref, bkj_ref,
               nb0_ref, nb1_ref, ro0_ref, ro1_ref, ms0_ref, ms1_ref):
    r4 = jnp.dot(rbf_ref[...], wr4_ref[...], preferred_element_type=jnp.float32)
    ga = ga_ref[...]
    gb = gb_ref[...]
    m_ji = _silu(ga[:, :D] + gb[:, :D] + r4[:, :D] + bji_ref[...])
    m_nb = _silu(ga[:, D:] + gb[:, D:] + r4[:, D:2 * D] + bkj_ref[...]) \
        * r4[:, 2 * D:3 * D]
    rbfo = r4[:, 3 * D:]
    mjis = rbfo * m_ji
    h = D // 2
    nb0_ref[...] = m_nb[:, :h]
    nb1_ref[...] = m_nb[:, h:]
    ro0_ref[...] = rbfo[:, :h]
    ro1_ref[...] = rbfo[:, h:]
    ms0_ref[...] = mjis[:, :h]
    ms1_ref[...] = mjis[:, h:]


def _tc24(rbf, ga, gb, wr4, bji, bkj):
    nblk = E // ROW_BLK
    full = lambda shp: pl.BlockSpec(shp, lambda i: (0, 0))
    half_spec = pl.BlockSpec((ROW_BLK, D // 2), lambda i: (i, 0))
    half_shape = jax.ShapeDtypeStruct((E, D // 2), jnp.float32)
    return pl.pallas_call(
        _tc24_body,
        grid=(nblk,),
        in_specs=[
            pl.BlockSpec((ROW_BLK, D), lambda i: (i, 0)),
            pl.BlockSpec((ROW_BLK, 2 * D), lambda i: (i, 0)),
            pl.BlockSpec((ROW_BLK, 2 * D), lambda i: (i, 0)),
            full((D, 4 * D)), full((1, D)), full((1, D)),
        ],
        out_specs=[half_spec] * 6,
        out_shape=[half_shape] * 6,
    )(rbf, ga, gb, wr4, bji, bkj)


def _tc5_body(x_ref, h_ref, a0_ref, a1_ref,
              wx2_ref, bx2_ref,
              w1a_ref, b1a_ref, w1b_ref, b1b_ref,
              w2a_ref, b2a_ref, w2b_ref, b2b_ref,
              w3a_ref, b3a_ref, w3b_ref, b3b_ref,
              wo1_ref, bo1_ref, wo2_ref, bo2_ref, wo3_ref, bo3_ref,
              wsc_ref, bsc_ref,
              hout_ref, s_ref):
    mm = lambda a, b: jnp.dot(a, b, preferred_element_type=jnp.float32)
    agg = jnp.concatenate([a0_ref[...], a1_ref[...]], axis=1)
    hh = h_ref[...] + agg
    hh = _silu(mm(hh, wx2_ref[...]) + bx2_ref[...])

    def res(zz, wa, ba, wb, bb):
        zo = _silu(mm(_silu(mm(zz, wa[...]) + ba[...]), wb[...]) + bb[...])
        return zo + zz

    hh = res(hh, w1a_ref, b1a_ref, w1b_ref, b1b_ref) + x_ref[...]
    hh = res(hh, w2a_ref, b2a_ref, w2b_ref, b2b_ref)
    hh = res(hh, w3a_ref, b3a_ref, w3b_ref, b3b_ref)
    hout_ref[...] = hh
    out = _silu(mm(hh, wo1_ref[...]) + bo1_ref[...])
    out = _silu(mm(out, wo2_ref[...]) + bo2_ref[...])
    out = _silu(mm(out, wo3_ref[...]) + bo3_ref[...])
    s_ref[...] = mm(out, wsc_ref[...]) + bsc_ref[...]


def _tc5(x, h, a0, a1, weights):
    nblk = N // ROW_BLK
    full = lambda shp: pl.BlockSpec(shp, lambda i: (0, 0))
    row = lambda w: pl.BlockSpec((ROW_BLK, w), lambda i: (i, 0))
    wspecs = []
    for w in weights:
        wspecs.append(full(w.shape))
    return pl.pallas_call(
        _tc5_body,
        grid=(nblk,),
        in_specs=[row(D), row(D)] + [row(D // 2)] * 2 + wspecs,
        out_specs=[row(D), row(D // 2)],
        out_shape=[
            jax.ShapeDtypeStruct((N, D), jnp.float32),
            jax.ShapeDtypeStruct((N, D // 2), jnp.float32),
        ],
    )(x, h, a0, a1, *weights)


from jax.experimental.pallas import tpu_sc as plsc

_NW = 32
_EPW = E // _NW
_K1 = 120
_NCH1 = _EPW // _K1
_TAIL1 = _EPW - _NCH1 * _K1


def _sc1_body(ti_h, tj_h, i_h, j_h, ga_h, gb_h,
              ii_v, jj_v, ba_v, bb_v, sa, sb):
    c = lax.axis_index("c")
    s = lax.axis_index("s")
    w = s * 2 + c
    base = w * _EPW

    def do_chunk(off, n):
        ii = ii_v if n == _K1 else ii_v.at[pl.ds(0, n)]
        jj = jj_v if n == _K1 else jj_v.at[pl.ds(0, n)]
        ba = ba_v if n == _K1 else ba_v.at[pl.ds(0, n)]
        bb = bb_v if n == _K1 else bb_v.at[pl.ds(0, n)]
        pltpu.sync_copy(i_h.at[pl.ds(off, n)], ii)
        pltpu.sync_copy(j_h.at[pl.ds(off, n)], jj)
        cpa = pltpu.async_copy(ti_h.at[ii], ba, sa)
        cpb = pltpu.async_copy(tj_h.at[jj], bb, sb)
        cpa.wait()
        cpb.wait()
        pltpu.sync_copy(ba, ga_h.at[pl.ds(off, n)])
        pltpu.sync_copy(bb, gb_h.at[pl.ds(off, n)])

    def step(ci, carry):
        do_chunk(base + ci * _K1, _K1)
        return carry

    lax.fori_loop(0, _NCH1, step, 0)
    do_chunk(base + _NCH1 * _K1, _TAIL1)


def _sc1(ti, tj, edge_index):
    mesh = plsc.VectorSubcoreMesh(core_axis_name="c", subcore_axis_name="s")
    f = functools.partial(
        pl.kernel,
        out_type=[
            jax.ShapeDtypeStruct((E, 2 * D), jnp.float32),
            jax.ShapeDtypeStruct((E, 2 * D), jnp.float32),
        ],
        mesh=mesh,
        scratch_types=[
            pltpu.VMEM((_K1,), jnp.int32),
            pltpu.VMEM((_K1,), jnp.int32),
            pltpu.VMEM((_K1, 2 * D), jnp.float32),
            pltpu.VMEM((_K1, 2 * D), jnp.float32),
            pltpu.SemaphoreType.DMA,
            pltpu.SemaphoreType.DMA,
        ],
    )(_sc1_body)
    return f(ti, tj, edge_index[1], edge_index[0])


_K2 = 80
_HD = D // 2
_NP = 10240
_RPS = _NP // 16


def _sc2_body(nb0_h, ro0_h, ms0_h, sh20_h, sh10_h,
              nb1_h, ro1_h, ms1_h, sh21_h, sh11_h,
              ikj_h, iji_h, d2_h, ijj_h, ijp_h, d1_h, ie_h,
              out_h,
              g1_v, g2_v, g3_v, i1_v, i2_v, dv_v, acc_sh,
              s1, s2, s3, si):
    c = lax.axis_index("c")
    s = lax.axis_index("s")

    def zrow(r, carry):
        for cc in range(_HD // 16):
            g1_v[r, pl.ds(cc * 16, 16)] = jnp.zeros((16,), jnp.float32)
        return carry
    lax.fori_loop(0, _K2, zrow, 0)

    def zcopy(k, carry):
        pltpu.sync_copy(g1_v, acc_sh.at[pl.ds(s * _RPS + k * _K2, _K2)])
        return carry
    lax.fori_loop(0, _RPS // _K2, zcopy, 0)
    plsc.subcore_barrier()

    def triplet_stream(nb_h, ro_h, idxa_h, idxb_h, dst_h, sh_h, nrows):
        nchunks = nrows // _K2
        niter = (nchunks + 15) // 16

        def step(g, carry):
            cid = g * 16 + s

            @pl.when(cid < nchunks)
            def _():
                off = cid * _K2
                ci1 = pltpu.async_copy(idxa_h.at[pl.ds(off, _K2)], i1_v, si)
                ci2 = pltpu.async_copy(idxb_h.at[pl.ds(off, _K2)], i2_v, si)
                ci3 = pltpu.async_copy(dst_h.at[pl.ds(off, _K2)], dv_v, si)
                cc_ = pltpu.async_copy(sh_h.at[pl.ds(off, _K2)], g3_v, s3)
                ci1.wait()
                ci2.wait()
                ci3.wait()
                ca = pltpu.async_copy(nb_h.at[i1_v], g1_v, s1)
                cb = pltpu.async_copy(ro_h.at[i2_v], g2_v, s2)
                ca.wait()
                cb.wait()
                cc_.wait()

                def mrow(r, carry2):
                    for cc in range(_HD // 16):
                        sl = pl.ds(cc * 16, 16)
                        g1_v[r, sl] = g1_v[r, sl] * g2_v[r, sl] * g3_v[r, sl]
                    return carry2
                lax.fori_loop(0, _K2, mrow, 0)
                pltpu.sync_copy(g1_v, acc_sh.at[dv_v], add=True)
            return carry
        lax.fori_loop(0, niter, step, 0)

    def edge_stream(ms_h):
        def estep(g, carry):
            off = (g * 16 + s) * _K2
            ci = pltpu.async_copy(ie_h.at[pl.ds(off, _K2)], dv_v, si)
            ce = pltpu.async_copy(ms_h.at[pl.ds(off, _K2)], g3_v, s3)
            ci.wait()
            ce.wait()
            pltpu.sync_copy(g3_v, acc_sh.at[dv_v], add=True)
            return carry
        lax.fori_loop(0, (E // _K2) // 16, estep, 0)

    @pl.when(c == 0)
    def _half0():
        triplet_stream(nb0_h, ro0_h, ikj_h, iji_h, d2_h, sh20_h, T2)
        triplet_stream(nb0_h, ro0_h, ijj_h, ijp_h, d1_h, sh10_h, T1)
        edge_stream(ms0_h)

    @pl.when(c == 1)
    def _half1():
        triplet_stream(nb1_h, ro1_h, ikj_h, iji_h, d2_h, sh21_h, T2)
        triplet_stream(nb1_h, ro1_h, ijj_h, ijp_h, d1_h, sh11_h, T1)
        edge_stream(ms1_h)

    plsc.subcore_barrier()
    pltpu.sync_copy(acc_sh.at[pl.ds(s * _RPS, _RPS)],
                    out_h.at[c, pl.ds(s * _RPS, _RPS)])


def _sc2(nb0, nb1, ro0, ro1, ms0, ms1, sh20, sh21, sh10, sh11,
         idx_kj, idx_ji, idx_jj_pair, idx_ji_pair, i_idx):
    dst2 = jnp.take(i_idx, idx_ji, axis=0)
    dst1 = jnp.take(i_idx, idx_ji_pair, axis=0)
    mesh = plsc.VectorSubcoreMesh(core_axis_name="c", subcore_axis_name="s")
    f = functools.partial(
        pl.kernel,
        out_type=jax.ShapeDtypeStruct((2, _NP, _HD), jnp.float32),
        mesh=mesh,
        scratch_types=[
            pltpu.VMEM((_K2, _HD), jnp.float32),
            pltpu.VMEM((_K2, _HD), jnp.float32),
            pltpu.VMEM((_K2, _HD), jnp.float32),
            pltpu.VMEM((_K2,), jnp.int32),
            pltpu.VMEM((_K2,), jnp.int32),
            pltpu.VMEM((_K2,), jnp.int32),
            pltpu.VMEM_SHARED((_NP, _HD), jnp.float32),
            pltpu.SemaphoreType.DMA,
            pltpu.SemaphoreType.DMA,
            pltpu.SemaphoreType.DMA,
            pltpu.SemaphoreType.DMA,
        ],
    )(_sc2_body)
    p = f(nb0, ro0, ms0, sh20, sh10, nb1, ro1, ms1, sh21, sh11,
          idx_kj, idx_ji, dst2, idx_jj_pair, idx_ji_pair, dst1, i_idx)
    return p[0, :N], p[1, :N]


def kernel(x, rbf, sbf2, sbf1, idx_kj, idx_ji, idx_jj_pair, idx_ji_pair,
           edge_index, params):
    p = params
    j_idx = edge_index[0]
    i_idx = edge_index[1]

    wti = jnp.concatenate([p['Wji'][:D], p['Wkj'][:D]], axis=1)
    wtj = jnp.concatenate([p['Wji'][D:2 * D], p['Wkj'][D:2 * D]], axis=1)
    wr4 = jnp.concatenate(
        [p['Wji'][2 * D:], p['Wkj'][2 * D:], p['Wrbf'], p['Wrbfo']], axis=1)
    wsc = jnp.concatenate(
        [p['Wout'], p['Watt'], jnp.zeros((D, D // 2 - 2), jnp.float32)], axis=1)
    bsc = jnp.concatenate(
        [p['bout'], jnp.zeros((D // 2 - 1,), jnp.float32)])[None]
    row_b = lambda b: b[None]

    h, ti, tj = _tc1(x, p['Wx1'], row_b(p['bx1']), wti, wtj)

    sh20, sh21 = _sbf_mlp(sbf2, p['Wsbf1'], row_b(p['bsbf1']),
                          p['Wsbf2'], row_b(p['bsbf2']))
    sh10, sh11 = _sbf_mlp(sbf1, p['Wsbf1'], row_b(p['bsbf1']),
                          p['Wsbf2'], row_b(p['bsbf2']))

    ga, gb = _sc1(ti, tj, edge_index)

    nb0, nb1, ro0, ro1, ms0, ms1 = _tc24(
        rbf, ga, gb, wr4, row_b(p['bji']), row_b(p['bkj']))

    a0, a1 = _sc2(nb0, nb1, ro0, ro1, ms0, ms1,
                  sh20, sh21, sh10, sh11,
                  idx_kj, idx_ji, idx_jj_pair, idx_ji_pair, i_idx)

    tc5_weights = [
        p['Wx2'], row_b(p['bx2']),
        p['Wr1a'], row_b(p['br1a']), p['Wr1b'], row_b(p['br1b']),
        p['Wr2a'], row_b(p['br2a']), p['Wr2b'], row_b(p['br2b']),
        p['Wr3a'], row_b(p['br3a']), p['Wr3b'], row_b(p['br3b']),
        p['Wo1'], row_b(p['bo1']), p['Wo2'], row_b(p['bo2']),
        p['Wo3'], row_b(p['bo3']),
        wsc, bsc,
    ]
    h_out, s = _tc5(x, h, a0, a1, tc5_weights)

    out_final = s[:, 0:1][None]
    att_score = s[:, 1:2][None]
    return (h_out, out_final, att_score)


# speedup vs baseline: 2.2708x; 1.1018x over previous
import functools

import jax
import jax.numpy as jnp
from jax import lax
from jax.experimental import pallas as pl
from jax.experimental.pallas import tpu as pltpu

D = 256
N = 10000
E = 160000
T2 = 160000
T1 = 80000

ROW_BLK = 1000


def _silu(v):
    return v * jax.nn.sigmoid(v)


def _tc1_body(x_ref, wx1_ref, bx1_ref, wti_ref, wtj_ref, h_ref, ti_ref, tj_ref):
    h = _silu(jnp.dot(x_ref[...], wx1_ref[...],
                      preferred_element_type=jnp.float32) + bx1_ref[...])
    h_ref[...] = h
    ti_ref[...] = jnp.dot(h, wti_ref[...], preferred_element_type=jnp.float32)
    tj_ref[...] = jnp.dot(h, wtj_ref[...], preferred_element_type=jnp.float32)


def _tc1(x, wx1, bx1, wti, wtj):
    nblk = N // ROW_BLK
    full = lambda shp: pl.BlockSpec(shp, lambda i: (0, 0))
    return pl.pallas_call(
        _tc1_body,
        grid=(nblk,),
        in_specs=[
            pl.BlockSpec((ROW_BLK, D), lambda i: (i, 0)),
            full((D, D)), full((1, D)), full((D, 2 * D)), full((D, 2 * D)),
        ],
        out_specs=[
            pl.BlockSpec((ROW_BLK, D), lambda i: (i, 0)),
            pl.BlockSpec((ROW_BLK, 2 * D), lambda i: (i, 0)),
            pl.BlockSpec((ROW_BLK, 2 * D), lambda i: (i, 0)),
        ],
        out_shape=[
            jax.ShapeDtypeStruct((N, D), jnp.float32),
            jax.ShapeDtypeStruct((N, 2 * D), jnp.float32),
            jax.ShapeDtypeStruct((N, 2 * D), jnp.float32),
        ],
    )(x, wx1, bx1, wti, wtj)


def _mlp2_body(x_ref, w1_ref, b1_ref, w2_ref, b2_ref, o0_ref, o1_ref):
    s = _silu(jnp.dot(x_ref[...], w1_ref[...],
                      preferred_element_type=jnp.float32) + b1_ref[...])
    o = _silu(jnp.dot(s, w2_ref[...],
                      preferred_element_type=jnp.float32) + b2_ref[...])
    o0_ref[...] = o[:, :D // 2]
    o1_ref[...] = o[:, D // 2:]


def _sbf_mlp(sbf, w1, b1, w2, b2):
    rows = sbf.shape[0]
    nblk = rows // ROW_BLK
    full = lambda shp: pl.BlockSpec(shp, lambda i: (0, 0))
    return pl.pallas_call(
        _mlp2_body,
        grid=(nblk,),
        in_specs=[
            pl.BlockSpec((ROW_BLK, D), lambda i: (i, 0)),
            full((D, D)), full((1, D)), full((D, D)), full((1, D)),
        ],
        out_specs=[
            pl.BlockSpec((ROW_BLK, D // 2), lambda i: (i, 0)),
            pl.BlockSpec((ROW_BLK, D // 2), lambda i: (i, 0)),
        ],
        out_shape=[
            jax.ShapeDtypeStruct((rows, D // 2), jnp.float32),
            jax.ShapeDtypeStruct((rows, D // 2), jnp.float32),
        ],
    )(sbf, w1, b1, w2, b2)


def _tc24_body(rbf_ref, ga_ref, gb_ref, wr4_ref, bji_ref, bkj_ref,
               nb0_ref, nb1_ref, ro0_ref, ro1_ref, ms0_ref, ms1_ref):
    r4 = jnp.dot(rbf_ref[...], wr4_ref[...], preferred_element_type=jnp.float32)
    ga = ga_ref[...]
    gb = gb_ref[...]
    m_ji = _silu(ga[:, :D] + gb[:, :D] + r4[:, :D] + bji_ref[...])
    m_nb = _silu(ga[:, D:] + gb[:, D:] + r4[:, D:2 * D] + bkj_ref[...]) \
        * r4[:, 2 * D:3 * D]
    rbfo = r4[:, 3 * D:]
    mjis = rbfo * m_ji
    h = D // 2
    nb0_ref[...] = m_nb[:, :h]
    nb1_ref[...] = m_nb[:, h:]
    ro0_ref[...] = rbfo[:, :h]
    ro1_ref[...] = rbfo[:, h:]
    ms0_ref[...] = mjis[:, :h]
    ms1_ref[...] = mjis[:, h:]


def _tc24(rbf, ga, gb, wr4, bji, bkj):
    nblk = E // ROW_BLK
    full = lambda shp: pl.BlockSpec(shp, lambda i: (0, 0))
    half_spec = pl.BlockSpec((ROW_BLK, D // 2), lambda i: (i, 0))
    half_shape = jax.ShapeDtypeStruct((E, D // 2), jnp.float32)
    return pl.pallas_call(
        _tc24_body,
        grid=(nblk,),
        in_specs=[
            pl.BlockSpec((ROW_BLK, D), lambda i: (i, 0)),
            pl.BlockSpec((ROW_BLK, 2 * D), lambda i: (i, 0)),
            pl.BlockSpec((ROW_BLK, 2 * D), lambda i: (i, 0)),
            full((D, 4 * D)), full((1, D)), full((1, D)),
        ],
        out_specs=[half_spec] * 6,
        out_shape=[half_shape] * 6,
    )(rbf, ga, gb, wr4, bji, bkj)


def _tc5_body(x_ref, h_ref, a0_ref, a1_ref,
              wx2_ref, bx2_ref,
              w1a_ref, b1a_ref, w1b_ref, b1b_ref,
              w2a_ref, b2a_ref, w2b_ref, b2b_ref,
              w3a_ref, b3a_ref, w3b_ref, b3b_ref,
              wo1_ref, bo1_ref, wo2_ref, bo2_ref, wo3_ref, bo3_ref,
              wsc_ref, bsc_ref,
              hout_ref, s_ref):
    mm = lambda a, b: jnp.dot(a, b, preferred_element_type=jnp.float32)
    agg = jnp.concatenate([a0_ref[...], a1_ref[...]], axis=1)
    hh = h_ref[...] + agg
    hh = _silu(mm(hh, wx2_ref[...]) + bx2_ref[...])

    def res(zz, wa, ba, wb, bb):
        zo = _silu(mm(_silu(mm(zz, wa[...]) + ba[...]), wb[...]) + bb[...])
        return zo + zz

    hh = res(hh, w1a_ref, b1a_ref, w1b_ref, b1b_ref) + x_ref[...]
    hh = res(hh, w2a_ref, b2a_ref, w2b_ref, b2b_ref)
    hh = res(hh, w3a_ref, b3a_ref, w3b_ref, b3b_ref)
    hout_ref[...] = hh
    out = _silu(mm(hh, wo1_ref[...]) + bo1_ref[...])
    out = _silu(mm(out, wo2_ref[...]) + bo2_ref[...])
    out = _silu(mm(out, wo3_ref[...]) + bo3_ref[...])
    s_ref[...] = mm(out, wsc_ref[...]) + bsc_ref[...]


def _tc5(x, h, a0, a1, weights):
    nblk = N // ROW_BLK
    full = lambda shp: pl.BlockSpec(shp, lambda i: (0, 0))
    row = lambda w: pl.BlockSpec((ROW_BLK, w), lambda i: (i, 0))
    wspecs = []
    for w in weights:
        wspecs.append(full(w.shape))
    return pl.pallas_call(
        _tc5_body,
        grid=(nblk,),
        in_specs=[row(D), row(D)] + [row(D // 2)] * 2 + wspecs,
        out_specs=[row(D), row(D // 2)],
        out_shape=[
            jax.ShapeDtypeStruct((N, D), jnp.float32),
            jax.ShapeDtypeStruct((N, D // 2), jnp.float32),
        ],
    )(x, h, a0, a1, *weights)


from jax.experimental.pallas import tpu_sc as plsc

_NW = 32
_EPW = E // _NW
_K1 = 120
_NCH1 = _EPW // _K1
_TAIL1 = _EPW - _NCH1 * _K1


def _sc1_body(ti_h, tj_h, i_h, j_h, ga_h, gb_h,
              ii_v, jj_v, ba_v, bb_v, sa, sb):
    c = lax.axis_index("c")
    s = lax.axis_index("s")
    w = s * 2 + c
    base = w * _EPW

    def do_chunk(off, n):
        ii = ii_v if n == _K1 else ii_v.at[pl.ds(0, n)]
        jj = jj_v if n == _K1 else jj_v.at[pl.ds(0, n)]
        ba = ba_v if n == _K1 else ba_v.at[pl.ds(0, n)]
        bb = bb_v if n == _K1 else bb_v.at[pl.ds(0, n)]
        pltpu.sync_copy(i_h.at[pl.ds(off, n)], ii)
        pltpu.sync_copy(j_h.at[pl.ds(off, n)], jj)
        cpa = pltpu.async_copy(ti_h.at[ii], ba, sa)
        cpb = pltpu.async_copy(tj_h.at[jj], bb, sb)
        cpa.wait()
        cpb.wait()
        pltpu.sync_copy(ba, ga_h.at[pl.ds(off, n)])
        pltpu.sync_copy(bb, gb_h.at[pl.ds(off, n)])

    def step(ci, carry):
        do_chunk(base + ci * _K1, _K1)
        return carry

    lax.fori_loop(0, _NCH1, step, 0)
    do_chunk(base + _NCH1 * _K1, _TAIL1)


def _sc1(ti, tj, edge_index):
    mesh = plsc.VectorSubcoreMesh(core_axis_name="c", subcore_axis_name="s")
    f = functools.partial(
        pl.kernel,
        out_type=[
            jax.ShapeDtypeStruct((E, 2 * D), jnp.float32),
            jax.ShapeDtypeStruct((E, 2 * D), jnp.float32),
        ],
        mesh=mesh,
        scratch_types=[
            pltpu.VMEM((_K1,), jnp.int32),
            pltpu.VMEM((_K1,), jnp.int32),
            pltpu.VMEM((_K1, 2 * D), jnp.float32),
            pltpu.VMEM((_K1, 2 * D), jnp.float32),
            pltpu.SemaphoreType.DMA,
            pltpu.SemaphoreType.DMA,
        ],
    )(_sc1_body)
    return f(ti, tj, edge_index[1], edge_index[0])


_K2 = 64
_HD = D // 2
_NP = 10112
_RPS = _NP // 16


def _sc2_body(nb0_h, ro0_h, ms0_h, sh20_h, sh10_h,
              nb1_h, ro1_h, ms1_h, sh21_h, sh11_h,
              ikj_h, iji_h, d2_h, ijj_h, ijp_h, d1_h, ie_h,
              out_h,
              g1_v, g2_v, g3_v, i1_v, i2_v, dv_v, acc_sh,
              sidx, sgat):
    c = lax.axis_index("c")
    s = lax.axis_index("s")

    def zrow(r, carry):
        for cc in range(_HD // 16):
            g1_v[0][r, pl.ds(cc * 16, 16)] = jnp.zeros((16,), jnp.float32)
        return carry
    lax.fori_loop(0, _K2, zrow, 0)

    def zcopy(k, carry):
        pltpu.sync_copy(g1_v[0], acc_sh.at[pl.ds(s * _RPS + k * _K2, _K2)])
        return carry
    lax.fori_loop(0, _RPS // _K2, zcopy, 0)
    pltpu.sync_copy(g1_v[0].at[pl.ds(0, _RPS % _K2)],
                    acc_sh.at[pl.ds(s * _RPS + (_RPS // _K2) * _K2,
                                    _RPS % _K2)])
    plsc.subcore_barrier()

    def triplet_stream(nb_h, ro_h, idxa_h, idxb_h, dst_h, sh_h, nrows):
        nchunks = nrows // _K2
        niter = (nchunks + 15) // 16

        def cid(t):
            return t * 16 + s

        def issue_idx(t, k):
            @pl.when(cid(t) < nchunks)
            def _():
                off = cid(t) * _K2
                pltpu.async_copy(idxa_h.at[pl.ds(off, _K2)], i1_v[k], sidx[k])
                pltpu.async_copy(idxb_h.at[pl.ds(off, _K2)], i2_v[k], sidx[k])
                pltpu.async_copy(dst_h.at[pl.ds(off, _K2)], dv_v[k], sidx[k])
                pltpu.async_copy(sh_h.at[pl.ds(off, _K2)], g3_v[k], sidx[k])

        def drain_idx(t, k):
            @pl.when(cid(t) < nchunks)
            def _():
                off = cid(t) * _K2
                pltpu.make_async_copy(idxa_h.at[pl.ds(off, _K2)], i1_v[k], sidx[k]).wait()
                pltpu.make_async_copy(idxb_h.at[pl.ds(off, _K2)], i2_v[k], sidx[k]).wait()
                pltpu.make_async_copy(dst_h.at[pl.ds(off, _K2)], dv_v[k], sidx[k]).wait()
                pltpu.make_async_copy(sh_h.at[pl.ds(off, _K2)], g3_v[k], sidx[k]).wait()

        def issue_gather(t, k):
            @pl.when(cid(t) < nchunks)
            def _():
                pltpu.async_copy(nb_h.at[i1_v[k]], g1_v[k], sgat[k])
                pltpu.async_copy(ro_h.at[i2_v[k]], g2_v[k], sgat[k])

        def drain_gather(t, k):
            @pl.when(cid(t) < nchunks)
            def _():
                pltpu.make_async_copy(nb_h.at[i1_v[k]], g1_v[k], sgat[k]).wait()
                pltpu.make_async_copy(ro_h.at[i2_v[k]], g2_v[k], sgat[k]).wait()

        def process(t, k):
            @pl.when(cid(t) < nchunks)
            def _():
                def mrow(r, carry2):
                    for cc in range(_HD // 16):
                        sl = pl.ds(cc * 16, 16)
                        g1_v[k][r, sl] = (g1_v[k][r, sl] * g2_v[k][r, sl]
                                          * g3_v[k][r, sl])
                    return carry2
                lax.fori_loop(0, _K2, mrow, 0)
                pltpu.sync_copy(g1_v[k], acc_sh.at[dv_v[k]], add=True)

        issue_idx(0, 0)
        drain_idx(0, 0)
        issue_gather(0, 0)
        issue_idx(1, 1)

        def step(q, carry):
            for p in range(2):
                t = 2 * q + p
                k = p
                drain_idx(t + 1, 1 - k)
                issue_gather(t + 1, 1 - k)
                drain_gather(t, k)
                process(t, k)
                issue_idx(t + 2, k)
            return carry
        lax.fori_loop(0, (niter + 1) // 2, step, 0)

    def edge_stream(ms_h):
        nchunks = E // _K2

        def cid(t):
            return t * 16 + s

        def issue(t, k):
            @pl.when(cid(t) < nchunks)
            def _():
                off = cid(t) * _K2
                pltpu.async_copy(ie_h.at[pl.ds(off, _K2)], dv_v[k], sidx[k])
                pltpu.async_copy(ms_h.at[pl.ds(off, _K2)], g3_v[k], sidx[k])

        def drain(t, k):
            @pl.when(cid(t) < nchunks)
            def _():
                off = cid(t) * _K2
                pltpu.make_async_copy(ie_h.at[pl.ds(off, _K2)], dv_v[k], sidx[k]).wait()
                pltpu.make_async_copy(ms_h.at[pl.ds(off, _K2)], g3_v[k], sidx[k]).wait()

        def scat(t, k):
            @pl.when(cid(t) < nchunks)
            def _():
                pltpu.sync_copy(g3_v[k], acc_sh.at[dv_v[k]], add=True)

        issue(0, 0)
        issue(1, 1)

        def estep(q, carry):
            for p in range(2):
                t = 2 * q + p
                k = p
                drain(t, k)
                scat(t, k)
                issue(t + 2, k)
            return carry
        niter = (nchunks + 15) // 16
        lax.fori_loop(0, (niter + 1) // 2, estep, 0)

    @pl.when(c == 0)
    def _half0():
        triplet_stream(nb0_h, ro0_h, ikj_h, iji_h, d2_h, sh20_h, T2)
        triplet_stream(nb0_h, ro0_h, ijj_h, ijp_h, d1_h, sh10_h, T1)
        edge_stream(ms0_h)

    @pl.when(c == 1)
    def _half1():
        triplet_stream(nb1_h, ro1_h, ikj_h, iji_h, d2_h, sh21_h, T2)
        triplet_stream(nb1_h, ro1_h, ijj_h, ijp_h, d1_h, sh11_h, T1)
        edge_stream(ms1_h)

    plsc.subcore_barrier()
    pltpu.sync_copy(acc_sh.at[pl.ds(s * _RPS, _RPS)],
                    out_h.at[c, pl.ds(s * _RPS, _RPS)])


def _sc2(nb0, nb1, ro0, ro1, ms0, ms1, sh20, sh21, sh10, sh11,
         idx_kj, idx_ji, idx_jj_pair, idx_ji_pair, i_idx):
    dst2 = jnp.take(i_idx, idx_ji, axis=0)
    dst1 = jnp.take(i_idx, idx_ji_pair, axis=0)
    mesh = plsc.VectorSubcoreMesh(core_axis_name="c", subcore_axis_name="s")
    f = functools.partial(
        pl.kernel,
        out_type=jax.ShapeDtypeStruct((2, _NP, _HD), jnp.float32),
        mesh=mesh,
        scratch_types=[
            [pltpu.VMEM((_K2, _HD), jnp.float32),
             pltpu.VMEM((_K2, _HD), jnp.float32)],
            [pltpu.VMEM((_K2, _HD), jnp.float32),
             pltpu.VMEM((_K2, _HD), jnp.float32)],
            [pltpu.VMEM((_K2, _HD), jnp.float32),
             pltpu.VMEM((_K2, _HD), jnp.float32)],
            [pltpu.VMEM((_K2,), jnp.int32), pltpu.VMEM((_K2,), jnp.int32)],
            [pltpu.VMEM((_K2,), jnp.int32), pltpu.VMEM((_K2,), jnp.int32)],
            [pltpu.VMEM((_K2,), jnp.int32), pltpu.VMEM((_K2,), jnp.int32)],
            pltpu.VMEM_SHARED((_NP, _HD), jnp.float32),
            [pltpu.SemaphoreType.DMA, pltpu.SemaphoreType.DMA],
            [pltpu.SemaphoreType.DMA, pltpu.SemaphoreType.DMA],
        ],
    )(_sc2_body)
    p = f(nb0, ro0, ms0, sh20, sh10, nb1, ro1, ms1, sh21, sh11,
          idx_kj, idx_ji, dst2, idx_jj_pair, idx_ji_pair, dst1, i_idx)
    return p[0, :N], p[1, :N]


def kernel(x, rbf, sbf2, sbf1, idx_kj, idx_ji, idx_jj_pair, idx_ji_pair,
           edge_index, params):
    p = params
    j_idx = edge_index[0]
    i_idx = edge_index[1]

    wti = jnp.concatenate([p['Wji'][:D], p['Wkj'][:D]], axis=1)
    wtj = jnp.concatenate([p['Wji'][D:2 * D], p['Wkj'][D:2 * D]], axis=1)
    wr4 = jnp.concatenate(
        [p['Wji'][2 * D:], p['Wkj'][2 * D:], p['Wrbf'], p['Wrbfo']], axis=1)
    wsc = jnp.concatenate(
        [p['Wout'], p['Watt'], jnp.zeros((D, D // 2 - 2), jnp.float32)], axis=1)
    bsc = jnp.concatenate(
        [p['bout'], jnp.zeros((D // 2 - 1,), jnp.float32)])[None]
    row_b = lambda b: b[None]

    h, ti, tj = _tc1(x, p['Wx1'], row_b(p['bx1']), wti, wtj)

    sh20, sh21 = _sbf_mlp(sbf2, p['Wsbf1'], row_b(p['bsbf1']),
                          p['Wsbf2'], row_b(p['bsbf2']))
    sh10, sh11 = _sbf_mlp(sbf1, p['Wsbf1'], row_b(p['bsbf1']),
                          p['Wsbf2'], row_b(p['bsbf2']))

    ga, gb = _sc1(ti, tj, edge_index)

    nb0, nb1, ro0, ro1, ms0, ms1 = _tc24(
        rbf, ga, gb, wr4, row_b(p['bji']), row_b(p['bkj']))

    a0, a1 = _sc2(nb0, nb1, ro0, ro1, ms0, ms1,
                  sh20, sh21, sh10, sh11,
                  idx_kj, idx_ji, idx_jj_pair, idx_ji_pair, i_idx)

    tc5_weights = [
        p['Wx2'], row_b(p['bx2']),
        p['Wr1a'], row_b(p['br1a']), p['Wr1b'], row_b(p['br1b']),
        p['Wr2a'], row_b(p['br2a']), p['Wr2b'], row_b(p['br2b']),
        p['Wr3a'], row_b(p['br3a']), p['Wr3b'], row_b(p['br3b']),
        p['Wo1'], row_b(p['bo1']), p['Wo2'], row_b(p['bo2']),
        p['Wo3'], row_b(p['bo3']),
        wsc, bsc,
    ]
    h_out, s = _tc5(x, h, a0, a1, tc5_weights)

    out_final = s[:, 0:1][None]
    att_score = s[:, 1:2][None]
    return (h_out, out_final, att_score)


# speedup vs baseline: 2.5982x; 1.1442x over previous
import functools

import jax
import jax.numpy as jnp
from jax import lax
from jax.experimental import pallas as pl
from jax.experimental.pallas import tpu as pltpu

D = 256
N = 10000
E = 160000
T2 = 160000
T1 = 80000

ROW_BLK = 1000


def _silu(v):
    return v * jax.nn.sigmoid(v)


def _tc1_body(x_ref, wx1_ref, bx1_ref, wti_ref, wtj_ref, h_ref, ti_ref, tj_ref):
    h = _silu(jnp.dot(x_ref[...], wx1_ref[...],
                      preferred_element_type=jnp.float32) + bx1_ref[...])
    h_ref[...] = h
    ti_ref[...] = jnp.dot(h, wti_ref[...], preferred_element_type=jnp.float32)
    tj_ref[...] = jnp.dot(h, wtj_ref[...], preferred_element_type=jnp.float32)


def _tc1(x, wx1, bx1, wti, wtj):
    nblk = N // ROW_BLK
    full = lambda shp: pl.BlockSpec(shp, lambda i: (0, 0))
    return pl.pallas_call(
        _tc1_body,
        grid=(nblk,),
        in_specs=[
            pl.BlockSpec((ROW_BLK, D), lambda i: (i, 0)),
            full((D, D)), full((1, D)), full((D, 2 * D)), full((D, 2 * D)),
        ],
        out_specs=[
            pl.BlockSpec((ROW_BLK, D), lambda i: (i, 0)),
            pl.BlockSpec((ROW_BLK, 2 * D), lambda i: (i, 0)),
            pl.BlockSpec((ROW_BLK, 2 * D), lambda i: (i, 0)),
        ],
        out_shape=[
            jax.ShapeDtypeStruct((N, D), jnp.float32),
            jax.ShapeDtypeStruct((N, 2 * D), jnp.float32),
            jax.ShapeDtypeStruct((N, 2 * D), jnp.float32),
        ],
    )(x, wx1, bx1, wti, wtj)


def _mlp2_body(x_ref, w1_ref, b1_ref, w2_ref, b2_ref, o0_ref, o1_ref):
    s = _silu(jnp.dot(x_ref[...], w1_ref[...],
                      preferred_element_type=jnp.float32) + b1_ref[...])
    o = _silu(jnp.dot(s, w2_ref[...],
                      preferred_element_type=jnp.float32) + b2_ref[...])
    o0_ref[...] = o[:, :D // 2]
    o1_ref[...] = o[:, D // 2:]


def _sbf_mlp(sbf, w1, b1, w2, b2):
    rows = sbf.shape[0]
    nblk = rows // ROW_BLK
    full = lambda shp: pl.BlockSpec(shp, lambda i: (0, 0))
    return pl.pallas_call(
        _mlp2_body,
        grid=(nblk,),
        in_specs=[
            pl.BlockSpec((ROW_BLK, D), lambda i: (i, 0)),
            full((D, D)), full((1, D)), full((D, D)), full((1, D)),
        ],
        out_specs=[
            pl.BlockSpec((ROW_BLK, D // 2), lambda i: (i, 0)),
            pl.BlockSpec((ROW_BLK, D // 2), lambda i: (i, 0)),
        ],
        out_shape=[
            jax.ShapeDtypeStruct((rows, D // 2), jnp.float32),
            jax.ShapeDtypeStruct((rows, D // 2), jnp.float32),
        ],
    )(sbf, w1, b1, w2, b2)


def _tc24_body(rbf_ref, pre_ref, wr4_ref, bji_ref, bkj_ref,
               nb0_ref, nb1_ref, ro0_ref, ro1_ref, ms0_ref, ms1_ref):
    r4 = jnp.dot(rbf_ref[...], wr4_ref[...], preferred_element_type=jnp.float32)
    pre = pre_ref[...]
    m_ji = _silu(pre[:, :D] + r4[:, :D] + bji_ref[...])
    m_nb = _silu(pre[:, D:] + r4[:, D:2 * D] + bkj_ref[...]) \
        * r4[:, 2 * D:3 * D]
    rbfo = r4[:, 3 * D:]
    mjis = rbfo * m_ji
    h = D // 2
    nb0_ref[...] = m_nb[:, :h]
    nb1_ref[...] = m_nb[:, h:]
    ro0_ref[...] = rbfo[:, :h]
    ro1_ref[...] = rbfo[:, h:]
    ms0_ref[...] = mjis[:, :h]
    ms1_ref[...] = mjis[:, h:]


def _tc24(rbf, pre, wr4, bji, bkj):
    nblk = E // ROW_BLK
    full = lambda shp: pl.BlockSpec(shp, lambda i: (0, 0))
    half_spec = pl.BlockSpec((ROW_BLK, D // 2), lambda i: (i, 0))
    half_shape = jax.ShapeDtypeStruct((E, D // 2), jnp.float32)
    return pl.pallas_call(
        _tc24_body,
        grid=(nblk,),
        in_specs=[
            pl.BlockSpec((ROW_BLK, D), lambda i: (i, 0)),
            pl.BlockSpec((ROW_BLK, 2 * D), lambda i: (i, 0)),
            full((D, 4 * D)), full((1, D)), full((1, D)),
        ],
        out_specs=[half_spec] * 6,
        out_shape=[half_shape] * 6,
    )(rbf, pre, wr4, bji, bkj)


def _tc5_body(x_ref, h_ref, a0_ref, a1_ref,
              wx2_ref, bx2_ref,
              w1a_ref, b1a_ref, w1b_ref, b1b_ref,
              w2a_ref, b2a_ref, w2b_ref, b2b_ref,
              w3a_ref, b3a_ref, w3b_ref, b3b_ref,
              wo1_ref, bo1_ref, wo2_ref, bo2_ref, wo3_ref, bo3_ref,
              wsc_ref, bsc_ref,
              hout_ref, s_ref):
    mm = lambda a, b: jnp.dot(a, b, preferred_element_type=jnp.float32)
    agg = jnp.concatenate([a0_ref[...], a1_ref[...]], axis=1)
    hh = h_ref[...] + agg
    hh = _silu(mm(hh, wx2_ref[...]) + bx2_ref[...])

    def res(zz, wa, ba, wb, bb):
        zo = _silu(mm(_silu(mm(zz, wa[...]) + ba[...]), wb[...]) + bb[...])
        return zo + zz

    hh = res(hh, w1a_ref, b1a_ref, w1b_ref, b1b_ref) + x_ref[...]
    hh = res(hh, w2a_ref, b2a_ref, w2b_ref, b2b_ref)
    hh = res(hh, w3a_ref, b3a_ref, w3b_ref, b3b_ref)
    hout_ref[...] = hh
    out = _silu(mm(hh, wo1_ref[...]) + bo1_ref[...])
    out = _silu(mm(out, wo2_ref[...]) + bo2_ref[...])
    out = _silu(mm(out, wo3_ref[...]) + bo3_ref[...])
    s_ref[...] = mm(out, wsc_ref[...]) + bsc_ref[...]


def _tc5(x, h, a0, a1, weights):
    nblk = N // ROW_BLK
    full = lambda shp: pl.BlockSpec(shp, lambda i: (0, 0))
    row = lambda w: pl.BlockSpec((ROW_BLK, w), lambda i: (i, 0))
    wspecs = []
    for w in weights:
        wspecs.append(full(w.shape))
    return pl.pallas_call(
        _tc5_body,
        grid=(nblk,),
        in_specs=[row(D), row(D)] + [row(D // 2)] * 2 + wspecs,
        out_specs=[row(D), row(D // 2)],
        out_shape=[
            jax.ShapeDtypeStruct((N, D), jnp.float32),
            jax.ShapeDtypeStruct((N, D // 2), jnp.float32),
        ],
    )(x, h, a0, a1, *weights)


from jax.experimental.pallas import tpu_sc as plsc

_NW = 32
_K1 = 40


def _sc1_body(ti_h, tj_h, i_h, j_h, pre_h,
              ii_v, jj_v, ba_v, bb_v, sidx, sgat, swr):
    c = lax.axis_index("c")
    s = lax.axis_index("s")
    w = s * 2 + c
    nchunks = E // _K1

    def cid(t):
        return t * _NW + w

    def issue_idx(t, k):
        @pl.when(cid(t) < nchunks)
        def _():
            off = cid(t) * _K1
            pltpu.async_copy(i_h.at[pl.ds(off, _K1)], ii_v[k], sidx[k])
            pltpu.async_copy(j_h.at[pl.ds(off, _K1)], jj_v[k], sidx[k])

    def drain_idx(t, k):
        @pl.when(cid(t) < nchunks)
        def _():
            off = cid(t) * _K1
            pltpu.make_async_copy(i_h.at[pl.ds(off, _K1)], ii_v[k], sidx[k]).wait()
            pltpu.make_async_copy(j_h.at[pl.ds(off, _K1)], jj_v[k], sidx[k]).wait()

    def issue_gather(t, k):
        @pl.when(cid(t) < nchunks)
        def _():
            pltpu.async_copy(ti_h.at[ii_v[k]], ba_v[k], sgat[k])
            pltpu.async_copy(tj_h.at[jj_v[k]], bb_v[k], sgat[k])

    def drain_gather(t, k):
        @pl.when(cid(t) < nchunks)
        def _():
            pltpu.make_async_copy(ti_h.at[ii_v[k]], ba_v[k], sgat[k]).wait()
            pltpu.make_async_copy(tj_h.at[jj_v[k]], bb_v[k], sgat[k]).wait()

    def add(t, k):
        @pl.when(cid(t) < nchunks)
        def _():
            def arow(r, carry2):
                for cc in range(2 * D // 16):
                    sl = pl.ds(cc * 16, 16)
                    ba_v[k][r, sl] = ba_v[k][r, sl] + bb_v[k][r, sl]
                return carry2
            lax.fori_loop(0, _K1, arow, 0)

    def issue_write(t, k):
        @pl.when(cid(t) < nchunks)
        def _():
            pltpu.async_copy(ba_v[k], pre_h.at[pl.ds(cid(t) * _K1, _K1)],
                             swr[k])

    def drain_write(t, k):
        @pl.when((cid(t) >= 0) & (cid(t) < nchunks))
        def _():
            pltpu.make_async_copy(ba_v[k], pre_h.at[pl.ds(cid(t) * _K1, _K1)],
                                  swr[k]).wait()

    issue_idx(0, 0)
    drain_idx(0, 0)
    issue_gather(0, 0)
    issue_idx(1, 1)

    niter = (nchunks + _NW - 1) // _NW
    nq = (niter + 1) // 2

    def step(q, carry):
        for p in range(2):
            t = 2 * q + p
            k = p
            drain_idx(t + 1, 1 - k)
            drain_write(t - 1, 1 - k)
            issue_gather(t + 1, 1 - k)
            drain_gather(t, k)
            add(t, k)
            issue_write(t, k)
            issue_idx(t + 2, k)
        return carry
    lax.fori_loop(0, nq, step, 0)
    drain_write(2 * nq - 1, 1)


def _sc1(ti, tj, edge_index):
    mesh = plsc.VectorSubcoreMesh(core_axis_name="c", subcore_axis_name="s")
    f = functools.partial(
        pl.kernel,
        out_type=jax.ShapeDtypeStruct((E, 2 * D), jnp.float32),
        mesh=mesh,
        scratch_types=[
            [pltpu.VMEM((_K1,), jnp.int32), pltpu.VMEM((_K1,), jnp.int32)],
            [pltpu.VMEM((_K1,), jnp.int32), pltpu.VMEM((_K1,), jnp.int32)],
            [pltpu.VMEM((_K1, 2 * D), jnp.float32),
             pltpu.VMEM((_K1, 2 * D), jnp.float32)],
            [pltpu.VMEM((_K1, 2 * D), jnp.float32),
             pltpu.VMEM((_K1, 2 * D), jnp.float32)],
            [pltpu.SemaphoreType.DMA, pltpu.SemaphoreType.DMA],
            [pltpu.SemaphoreType.DMA, pltpu.SemaphoreType.DMA],
            [pltpu.SemaphoreType.DMA, pltpu.SemaphoreType.DMA],
        ],
    )(_sc1_body)
    return f(ti, tj, edge_index[1], edge_index[0])


_K2 = 64
_HD = D // 2
_NP = 10112
_RPS = _NP // 16


def _sc2_body(nb0_h, ro0_h, ms0_h, sh20_h, sh10_h,
              nb1_h, ro1_h, ms1_h, sh21_h, sh11_h,
              ikj_h, iji_h, d2_h, ijj_h, ijp_h, d1_h, ie_h,
              out_h,
              g1_v, g2_v, g3_v, i1_v, i2_v, dv_v, acc_sh,
              sidx, sgat):
    c = lax.axis_index("c")
    s = lax.axis_index("s")

    def zrow(r, carry):
        for cc in range(_HD // 16):
            g1_v[0][r, pl.ds(cc * 16, 16)] = jnp.zeros((16,), jnp.float32)
        return carry
    lax.fori_loop(0, _K2, zrow, 0)

    def zcopy(k, carry):
        pltpu.sync_copy(g1_v[0], acc_sh.at[pl.ds(s * _RPS + k * _K2, _K2)])
        return carry
    lax.fori_loop(0, _RPS // _K2, zcopy, 0)
    pltpu.sync_copy(g1_v[0].at[pl.ds(0, _RPS % _K2)],
                    acc_sh.at[pl.ds(s * _RPS + (_RPS // _K2) * _K2,
                                    _RPS % _K2)])
    plsc.subcore_barrier()

    def triplet_stream(nb_h, ro_h, idxa_h, idxb_h, dst_h, sh_h, nrows):
        nchunks = nrows // _K2
        niter = (nchunks + 15) // 16

        def cid(t):
            return t * 16 + s

        def issue_idx(t, k):
            @pl.when(cid(t) < nchunks)
            def _():
                off = cid(t) * _K2
                pltpu.async_copy(idxa_h.at[pl.ds(off, _K2)], i1_v[k], sidx[k])
                pltpu.async_copy(idxb_h.at[pl.ds(off, _K2)], i2_v[k], sidx[k])
                pltpu.async_copy(dst_h.at[pl.ds(off, _K2)], dv_v[k], sidx[k])
                pltpu.async_copy(sh_h.at[pl.ds(off, _K2)], g3_v[k], sidx[k])

        def drain_idx(t, k):
            @pl.when(cid(t) < nchunks)
            def _():
                off = cid(t) * _K2
                pltpu.make_async_copy(idxa_h.at[pl.ds(off, _K2)], i1_v[k], sidx[k]).wait()
                pltpu.make_async_copy(idxb_h.at[pl.ds(off, _K2)], i2_v[k], sidx[k]).wait()
                pltpu.make_async_copy(dst_h.at[pl.ds(off, _K2)], dv_v[k], sidx[k]).wait()
                pltpu.make_async_copy(sh_h.at[pl.ds(off, _K2)], g3_v[k], sidx[k]).wait()

        def issue_gather(t, k):
            @pl.when(cid(t) < nchunks)
            def _():
                pltpu.async_copy(nb_h.at[i1_v[k]], g1_v[k], sgat[k])
                pltpu.async_copy(ro_h.at[i2_v[k]], g2_v[k], sgat[k])

        def drain_gather(t, k):
            @pl.when(cid(t) < nchunks)
            def _():
                pltpu.make_async_copy(nb_h.at[i1_v[k]], g1_v[k], sgat[k]).wait()
                pltpu.make_async_copy(ro_h.at[i2_v[k]], g2_v[k], sgat[k]).wait()

        def process(t, k):
            @pl.when(cid(t) < nchunks)
            def _():
                def mrow(r, carry2):
                    for cc in range(_HD // 16):
                        sl = pl.ds(cc * 16, 16)
                        g1_v[k][r, sl] = (g1_v[k][r, sl] * g2_v[k][r, sl]
                                          * g3_v[k][r, sl])
                    return carry2
                lax.fori_loop(0, _K2, mrow, 0)
                pltpu.sync_copy(g1_v[k], acc_sh.at[dv_v[k]], add=True)

        issue_idx(0, 0)
        drain_idx(0, 0)
        issue_gather(0, 0)
        issue_idx(1, 1)

        def step(q, carry):
            for p in range(2):
                t = 2 * q + p
                k = p
                drain_idx(t + 1, 1 - k)
                issue_gather(t + 1, 1 - k)
                drain_gather(t, k)
                process(t, k)
                issue_idx(t + 2, k)
            return carry
        lax.fori_loop(0, (niter + 1) // 2, step, 0)

    def edge_stream(ms_h):
        nchunks = E // _K2

        def cid(t):
            return t * 16 + s

        def issue(t, k):
            @pl.when(cid(t) < nchunks)
            def _():
                off = cid(t) * _K2
                pltpu.async_copy(ie_h.at[pl.ds(off, _K2)], dv_v[k], sidx[k])
                pltpu.async_copy(ms_h.at[pl.ds(off, _K2)], g3_v[k], sidx[k])

        def drain(t, k):
            @pl.when(cid(t) < nchunks)
            def _():
                off = cid(t) * _K2
                pltpu.make_async_copy(ie_h.at[pl.ds(off, _K2)], dv_v[k], sidx[k]).wait()
                pltpu.make_async_copy(ms_h.at[pl.ds(off, _K2)], g3_v[k], sidx[k]).wait()

        def scat(t, k):
            @pl.when(cid(t) < nchunks)
            def _():
                pltpu.sync_copy(g3_v[k], acc_sh.at[dv_v[k]], add=True)

        issue(0, 0)
        issue(1, 1)

        def estep(q, carry):
            for p in range(2):
                t = 2 * q + p
                k = p
                drain(t, k)
                scat(t, k)
                issue(t + 2, k)
            return carry
        niter = (nchunks + 15) // 16
        lax.fori_loop(0, (niter + 1) // 2, estep, 0)

    @pl.when(c == 0)
    def _half0():
        triplet_stream(nb0_h, ro0_h, ikj_h, iji_h, d2_h, sh20_h, T2)
        triplet_stream(nb0_h, ro0_h, ijj_h, ijp_h, d1_h, sh10_h, T1)
        edge_stream(ms0_h)

    @pl.when(c == 1)
    def _half1():
        triplet_stream(nb1_h, ro1_h, ikj_h, iji_h, d2_h, sh21_h, T2)
        triplet_stream(nb1_h, ro1_h, ijj_h, ijp_h, d1_h, sh11_h, T1)
        edge_stream(ms1_h)

    plsc.subcore_barrier()
    pltpu.sync_copy(acc_sh.at[pl.ds(s * _RPS, _RPS)],
                    out_h.at[c, pl.ds(s * _RPS, _RPS)])


def _sc2(nb0, nb1, ro0, ro1, ms0, ms1, sh20, sh21, sh10, sh11,
         idx_kj, idx_ji, idx_jj_pair, idx_ji_pair, i_idx):
    dst2 = jnp.take(i_idx, idx_ji, axis=0)
    dst1 = jnp.take(i_idx, idx_ji_pair, axis=0)
    mesh = plsc.VectorSubcoreMesh(core_axis_name="c", subcore_axis_name="s")
    f = functools.partial(
        pl.kernel,
        out_type=jax.ShapeDtypeStruct((2, _NP, _HD), jnp.float32),
        mesh=mesh,
        scratch_types=[
            [pltpu.VMEM((_K2, _HD), jnp.float32),
             pltpu.VMEM((_K2, _HD), jnp.float32)],
            [pltpu.VMEM((_K2, _HD), jnp.float32),
             pltpu.VMEM((_K2, _HD), jnp.float32)],
            [pltpu.VMEM((_K2, _HD), jnp.float32),
             pltpu.VMEM((_K2, _HD), jnp.float32)],
            [pltpu.VMEM((_K2,), jnp.int32), pltpu.VMEM((_K2,), jnp.int32)],
            [pltpu.VMEM((_K2,), jnp.int32), pltpu.VMEM((_K2,), jnp.int32)],
            [pltpu.VMEM((_K2,), jnp.int32), pltpu.VMEM((_K2,), jnp.int32)],
            pltpu.VMEM_SHARED((_NP, _HD), jnp.float32),
            [pltpu.SemaphoreType.DMA, pltpu.SemaphoreType.DMA],
            [pltpu.SemaphoreType.DMA, pltpu.SemaphoreType.DMA],
        ],
    )(_sc2_body)
    p = f(nb0, ro0, ms0, sh20, sh10, nb1, ro1, ms1, sh21, sh11,
          idx_kj, idx_ji, dst2, idx_jj_pair, idx_ji_pair, dst1, i_idx)
    return p[0, :N], p[1, :N]


def kernel(x, rbf, sbf2, sbf1, idx_kj, idx_ji, idx_jj_pair, idx_ji_pair,
           edge_index, params):
    p = params
    j_idx = edge_index[0]
    i_idx = edge_index[1]

    wti = jnp.concatenate([p['Wji'][:D], p['Wkj'][:D]], axis=1)
    wtj = jnp.concatenate([p['Wji'][D:2 * D], p['Wkj'][D:2 * D]], axis=1)
    wr4 = jnp.concatenate(
        [p['Wji'][2 * D:], p['Wkj'][2 * D:], p['Wrbf'], p['Wrbfo']], axis=1)
    wsc = jnp.concatenate(
        [p['Wout'], p['Watt'], jnp.zeros((D, D // 2 - 2), jnp.float32)], axis=1)
    bsc = jnp.concatenate(
        [p['bout'], jnp.zeros((D // 2 - 1,), jnp.float32)])[None]
    row_b = lambda b: b[None]

    h, ti, tj = _tc1(x, p['Wx1'], row_b(p['bx1']), wti, wtj)

    sh20, sh21 = _sbf_mlp(sbf2, p['Wsbf1'], row_b(p['bsbf1']),
                          p['Wsbf2'], row_b(p['bsbf2']))
    sh10, sh11 = _sbf_mlp(sbf1, p['Wsbf1'], row_b(p['bsbf1']),
                          p['Wsbf2'], row_b(p['bsbf2']))

    pre = _sc1(ti, tj, edge_index)

    nb0, nb1, ro0, ro1, ms0, ms1 = _tc24(
        rbf, pre, wr4, row_b(p['bji']), row_b(p['bkj']))

    a0, a1 = _sc2(nb0, nb1, ro0, ro1, ms0, ms1,
                  sh20, sh21, sh10, sh11,
                  idx_kj, idx_ji, idx_jj_pair, idx_ji_pair, i_idx)

    tc5_weights = [
        p['Wx2'], row_b(p['bx2']),
        p['Wr1a'], row_b(p['br1a']), p['Wr1b'], row_b(p['br1b']),
        p['Wr2a'], row_b(p['br2a']), p['Wr2b'], row_b(p['br2b']),
        p['Wr3a'], row_b(p['br3a']), p['Wr3b'], row_b(p['br3b']),
        p['Wo1'], row_b(p['bo1']), p['Wo2'], row_b(p['bo2']),
        p['Wo3'], row_b(p['bo3']),
        wsc, bsc,
    ]
    h_out, s = _tc5(x, h, a0, a1, tc5_weights)

    out_final = s[:, 0:1][None]
    att_score = s[:, 1:2][None]
    return (h_out, out_final, att_score)


# speedup vs baseline: 2.7226x; 1.0479x over previous
import functools

import jax
import jax.numpy as jnp
from jax import lax
from jax.experimental import pallas as pl
from jax.experimental.pallas import tpu as pltpu

D = 256
N = 10000
E = 160000
T2 = 160000
T1 = 80000

ROW_BLK = 1000


def _silu(v):
    return v * jax.nn.sigmoid(v)


def _pack_bf16_pair(lo_f32, hi_f32):
    rnd = jnp.int32(0x8000)
    lo_b = lax.bitcast_convert_type(lo_f32, jnp.int32)
    hi_b = lax.bitcast_convert_type(hi_f32, jnp.int32)
    lo16 = ((lo_b + rnd) >> 16) & jnp.int32(0xFFFF)
    hi16 = (hi_b + rnd) & jnp.int32(-65536)
    return hi16 | lo16


def _tc1_body(x_ref, wx1_ref, bx1_ref, wti_ref, wtj_ref, h_ref, ti_ref, tj_ref):
    h = _silu(jnp.dot(x_ref[...], wx1_ref[...],
                      preferred_element_type=jnp.float32) + bx1_ref[...])
    h_ref[...] = h
    ti = jnp.dot(h, wti_ref[...], preferred_element_type=jnp.float32)
    tj = jnp.dot(h, wtj_ref[...], preferred_element_type=jnp.float32)
    ti_ref[...] = _pack_bf16_pair(ti[:, :D], ti[:, D:])
    tj_ref[...] = _pack_bf16_pair(tj[:, :D], tj[:, D:])


def _tc1(x, wx1, bx1, wti, wtj):
    nblk = N // ROW_BLK
    full = lambda shp: pl.BlockSpec(shp, lambda i: (0, 0))
    return pl.pallas_call(
        _tc1_body,
        grid=(nblk,),
        in_specs=[
            pl.BlockSpec((ROW_BLK, D), lambda i: (i, 0)),
            full((D, D)), full((1, D)), full((D, 2 * D)), full((D, 2 * D)),
        ],
        out_specs=[
            pl.BlockSpec((ROW_BLK, D), lambda i: (i, 0)),
            pl.BlockSpec((ROW_BLK, D), lambda i: (i, 0)),
            pl.BlockSpec((ROW_BLK, D), lambda i: (i, 0)),
        ],
        out_shape=[
            jax.ShapeDtypeStruct((N, D), jnp.float32),
            jax.ShapeDtypeStruct((N, D), jnp.int32),
            jax.ShapeDtypeStruct((N, D), jnp.int32),
        ],
    )(x, wx1, bx1, wti, wtj)


def _mlp2_body(x_ref, w1_ref, b1_ref, w2_ref, b2_ref, o0_ref, o1_ref):
    s = _silu(jnp.dot(x_ref[...], w1_ref[...],
                      preferred_element_type=jnp.float32) + b1_ref[...])
    o = _silu(jnp.dot(s, w2_ref[...],
                      preferred_element_type=jnp.float32) + b2_ref[...])
    o0_ref[...] = o[:, :D // 2]
    o1_ref[...] = o[:, D // 2:]


def _sbf_mlp(sbf, w1, b1, w2, b2):
    rows = sbf.shape[0]
    nblk = rows // ROW_BLK
    full = lambda shp: pl.BlockSpec(shp, lambda i: (0, 0))
    return pl.pallas_call(
        _mlp2_body,
        grid=(nblk,),
        in_specs=[
            pl.BlockSpec((ROW_BLK, D), lambda i: (i, 0)),
            full((D, D)), full((1, D)), full((D, D)), full((1, D)),
        ],
        out_specs=[
            pl.BlockSpec((ROW_BLK, D // 2), lambda i: (i, 0)),
            pl.BlockSpec((ROW_BLK, D // 2), lambda i: (i, 0)),
        ],
        out_shape=[
            jax.ShapeDtypeStruct((rows, D // 2), jnp.float32),
            jax.ShapeDtypeStruct((rows, D // 2), jnp.float32),
        ],
    )(sbf, w1, b1, w2, b2)


def _tc24_body(rbf_ref, ga_ref, gb_ref, wr4_ref, bji_ref, bkj_ref,
               nb0_ref, nb1_ref, ro0_ref, ro1_ref, ms0_ref, ms1_ref):
    r4 = jnp.dot(rbf_ref[...], wr4_ref[...], preferred_element_type=jnp.float32)
    ga = ga_ref[...]
    gb = gb_ref[...]
    unlo = lambda w: lax.bitcast_convert_type(w << 16, jnp.float32)
    unhi = lambda w: lax.bitcast_convert_type(w & jnp.int32(-65536), jnp.float32)
    pre_ji = unlo(ga) + unlo(gb)
    pre_kj = unhi(ga) + unhi(gb)
    m_ji = _silu(pre_ji + r4[:, :D] + bji_ref[...])
    m_nb = _silu(pre_kj + r4[:, D:2 * D] + bkj_ref[...]) \
        * r4[:, 2 * D:3 * D]
    rbfo = r4[:, 3 * D:]
    mjis = rbfo * m_ji
    h = D // 2
    nb0_ref[...] = m_nb[:, :h]
    nb1_ref[...] = m_nb[:, h:]
    ro0_ref[...] = rbfo[:, :h]
    ro1_ref[...] = rbfo[:, h:]
    ms0_ref[...] = mjis[:, :h]
    ms1_ref[...] = mjis[:, h:]


def _tc24(rbf, ga, gb, wr4, bji, bkj):
    nblk = E // ROW_BLK
    full = lambda shp: pl.BlockSpec(shp, lambda i: (0, 0))
    half_spec = pl.BlockSpec((ROW_BLK, D // 2), lambda i: (i, 0))
    half_shape = jax.ShapeDtypeStruct((E, D // 2), jnp.float32)
    return pl.pallas_call(
        _tc24_body,
        grid=(nblk,),
        in_specs=[
            pl.BlockSpec((ROW_BLK, D), lambda i: (i, 0)),
            pl.BlockSpec((ROW_BLK, D), lambda i: (i, 0)),
            pl.BlockSpec((ROW_BLK, D), lambda i: (i, 0)),
            full((D, 4 * D)), full((1, D)), full((1, D)),
        ],
        out_specs=[half_spec] * 6,
        out_shape=[half_shape] * 6,
    )(rbf, ga, gb, wr4, bji, bkj)


def _tc5_body(x_ref, h_ref, a0_ref, a1_ref,
              wx2_ref, bx2_ref,
              w1a_ref, b1a_ref, w1b_ref, b1b_ref,
              w2a_ref, b2a_ref, w2b_ref, b2b_ref,
              w3a_ref, b3a_ref, w3b_ref, b3b_ref,
              wo1_ref, bo1_ref, wo2_ref, bo2_ref, wo3_ref, bo3_ref,
              wsc_ref, bsc_ref,
              hout_ref, s_ref):
    mm = lambda a, b: jnp.dot(a, b, preferred_element_type=jnp.float32)
    agg = jnp.concatenate([a0_ref[...], a1_ref[...]], axis=1)
    hh = h_ref[...] + agg
    hh = _silu(mm(hh, wx2_ref[...]) + bx2_ref[...])

    def res(zz, wa, ba, wb, bb):
        zo = _silu(mm(_silu(mm(zz, wa[...]) + ba[...]), wb[...]) + bb[...])
        return zo + zz

    hh = res(hh, w1a_ref, b1a_ref, w1b_ref, b1b_ref) + x_ref[...]
    hh = res(hh, w2a_ref, b2a_ref, w2b_ref, b2b_ref)
    hh = res(hh, w3a_ref, b3a_ref, w3b_ref, b3b_ref)
    hout_ref[...] = hh
    out = _silu(mm(hh, wo1_ref[...]) + bo1_ref[...])
    out = _silu(mm(out, wo2_ref[...]) + bo2_ref[...])
    out = _silu(mm(out, wo3_ref[...]) + bo3_ref[...])
    s_ref[...] = mm(out, wsc_ref[...]) + bsc_ref[...]


def _tc5(x, h, a0, a1, weights):
    nblk = N // ROW_BLK
    full = lambda shp: pl.BlockSpec(shp, lambda i: (0, 0))
    row = lambda w: pl.BlockSpec((ROW_BLK, w), lambda i: (i, 0))
    wspecs = []
    for w in weights:
        wspecs.append(full(w.shape))
    return pl.pallas_call(
        _tc5_body,
        grid=(nblk,),
        in_specs=[row(D), row(D)] + [row(D // 2)] * 2 + wspecs,
        out_specs=[row(D), row(D // 2)],
        out_shape=[
            jax.ShapeDtypeStruct((N, D), jnp.float32),
            jax.ShapeDtypeStruct((N, D // 2), jnp.float32),
        ],
    )(x, h, a0, a1, *weights)


from jax.experimental.pallas import tpu_sc as plsc

_NW = 32
_K1 = 40


def _sc1_body(ti_h, tj_h, i_h, j_h, ga_h, gb_h,
              ii_v, jj_v, ba_v, bb_v, sidx, sgat, swr):
    c = lax.axis_index("c")
    s = lax.axis_index("s")
    w = s * 2 + c
    nchunks = E // _K1

    def cid(t):
        return t * _NW + w

    def issue_idx(t, k):
        @pl.when(cid(t) < nchunks)
        def _():
            off = cid(t) * _K1
            pltpu.async_copy(i_h.at[pl.ds(off, _K1)], ii_v[k], sidx[k])
            pltpu.async_copy(j_h.at[pl.ds(off, _K1)], jj_v[k], sidx[k])

    def drain_idx(t, k):
        @pl.when(cid(t) < nchunks)
        def _():
            off = cid(t) * _K1
            pltpu.make_async_copy(i_h.at[pl.ds(off, _K1)], ii_v[k], sidx[k]).wait()
            pltpu.make_async_copy(j_h.at[pl.ds(off, _K1)], jj_v[k], sidx[k]).wait()

    def issue_gather(t, k):
        @pl.when(cid(t) < nchunks)
        def _():
            pltpu.async_copy(ti_h.at[ii_v[k]], ba_v[k], sgat[k])
            pltpu.async_copy(tj_h.at[jj_v[k]], bb_v[k], sgat[k])

    def drain_gather(t, k):
        @pl.when(cid(t) < nchunks)
        def _():
            pltpu.make_async_copy(ti_h.at[ii_v[k]], ba_v[k], sgat[k]).wait()
            pltpu.make_async_copy(tj_h.at[jj_v[k]], bb_v[k], sgat[k]).wait()

    def issue_write(t, k):
        @pl.when(cid(t) < nchunks)
        def _():
            pltpu.async_copy(ba_v[k], ga_h.at[pl.ds(cid(t) * _K1, _K1)],
                             swr[k])
            pltpu.async_copy(bb_v[k], gb_h.at[pl.ds(cid(t) * _K1, _K1)],
                             swr[k])

    def drain_write(t, k):
        @pl.when((cid(t) >= 0) & (cid(t) < nchunks))
        def _():
            pltpu.make_async_copy(ba_v[k], ga_h.at[pl.ds(cid(t) * _K1, _K1)],
                                  swr[k]).wait()
            pltpu.make_async_copy(bb_v[k], gb_h.at[pl.ds(cid(t) * _K1, _K1)],
                                  swr[k]).wait()

    issue_idx(0, 0)
    drain_idx(0, 0)
    issue_gather(0, 0)
    issue_idx(1, 1)

    niter = (nchunks + _NW - 1) // _NW
    nq = (niter + 1) // 2

    def step(q, carry):
        for p in range(2):
            t = 2 * q + p
            k = p
            drain_idx(t + 1, 1 - k)
            drain_write(t - 1, 1 - k)
            issue_gather(t + 1, 1 - k)
            drain_gather(t, k)
            issue_write(t, k)
            issue_idx(t + 2, k)
        return carry
    lax.fori_loop(0, nq, step, 0)
    drain_write(2 * nq - 1, 1)


def _sc1(ti, tj, edge_index):
    mesh = plsc.VectorSubcoreMesh(core_axis_name="c", subcore_axis_name="s")
    f = functools.partial(
        pl.kernel,
        out_type=[jax.ShapeDtypeStruct((E, D), jnp.int32),
                  jax.ShapeDtypeStruct((E, D), jnp.int32)],
        mesh=mesh,
        scratch_types=[
            [pltpu.VMEM((_K1,), jnp.int32), pltpu.VMEM((_K1,), jnp.int32)],
            [pltpu.VMEM((_K1,), jnp.int32), pltpu.VMEM((_K1,), jnp.int32)],
            [pltpu.VMEM((_K1, D), jnp.int32), pltpu.VMEM((_K1, D), jnp.int32)],
            [pltpu.VMEM((_K1, D), jnp.int32), pltpu.VMEM((_K1, D), jnp.int32)],
            [pltpu.SemaphoreType.DMA, pltpu.SemaphoreType.DMA],
            [pltpu.SemaphoreType.DMA, pltpu.SemaphoreType.DMA],
            [pltpu.SemaphoreType.DMA, pltpu.SemaphoreType.DMA],
        ],
    )(_sc1_body)
    return f(ti, tj, edge_index[1], edge_index[0])


_K2 = 64
_HD = D // 2
_NP = 10112
_RPS = _NP // 16


def _sc2_body(nb0_h, ro0_h, ms0_h, sh20_h, sh10_h,
              nb1_h, ro1_h, ms1_h, sh21_h, sh11_h,
              ikj_h, iji_h, d2_h, ijj_h, ijp_h, d1_h, ie_h,
              out_h,
              g1_v, g2_v, g3_v, i1_v, i2_v, dv_v, acc_sh,
              sidx, sgat):
    c = lax.axis_index("c")
    s = lax.axis_index("s")

    def zrow(r, carry):
        for cc in range(_HD // 16):
            g1_v[0][r, pl.ds(cc * 16, 16)] = jnp.zeros((16,), jnp.float32)
        return carry
    lax.fori_loop(0, _K2, zrow, 0)

    def zcopy(k, carry):
        pltpu.sync_copy(g1_v[0], acc_sh.at[pl.ds(s * _RPS + k * _K2, _K2)])
        return carry
    lax.fori_loop(0, _RPS // _K2, zcopy, 0)
    pltpu.sync_copy(g1_v[0].at[pl.ds(0, _RPS % _K2)],
                    acc_sh.at[pl.ds(s * _RPS + (_RPS // _K2) * _K2,
                                    _RPS % _K2)])
    plsc.subcore_barrier()

    def triplet_stream(nb_h, ro_h, idxa_h, idxb_h, dst_h, sh_h, nrows):
        nchunks = nrows // _K2
        niter = (nchunks + 15) // 16

        def cid(t):
            return t * 16 + s

        def issue_idx(t, k):
            @pl.when(cid(t) < nchunks)
            def _():
                off = cid(t) * _K2
                pltpu.async_copy(idxa_h.at[pl.ds(off, _K2)], i1_v[k], sidx[k])
                pltpu.async_copy(idxb_h.at[pl.ds(off, _K2)], i2_v[k], sidx[k])
                pltpu.async_copy(dst_h.at[pl.ds(off, _K2)], dv_v[k], sidx[k])
                pltpu.async_copy(sh_h.at[pl.ds(off, _K2)], g3_v[k], sidx[k])

        def drain_idx(t, k):
            @pl.when(cid(t) < nchunks)
            def _():
                off = cid(t) * _K2
                pltpu.make_async_copy(idxa_h.at[pl.ds(off, _K2)], i1_v[k], sidx[k]).wait()
                pltpu.make_async_copy(idxb_h.at[pl.ds(off, _K2)], i2_v[k], sidx[k]).wait()
                pltpu.make_async_copy(dst_h.at[pl.ds(off, _K2)], dv_v[k], sidx[k]).wait()
                pltpu.make_async_copy(sh_h.at[pl.ds(off, _K2)], g3_v[k], sidx[k]).wait()

        def issue_gather(t, k):
            @pl.when(cid(t) < nchunks)
            def _():
                pltpu.async_copy(nb_h.at[i1_v[k]], g1_v[k], sgat[k])
                pltpu.async_copy(ro_h.at[i2_v[k]], g2_v[k], sgat[k])

        def drain_gather(t, k):
            @pl.when(cid(t) < nchunks)
            def _():
                pltpu.make_async_copy(nb_h.at[i1_v[k]], g1_v[k], sgat[k]).wait()
                pltpu.make_async_copy(ro_h.at[i2_v[k]], g2_v[k], sgat[k]).wait()

        def process(t, k):
            @pl.when(cid(t) < nchunks)
            def _():
                def mrow(r, carry2):
                    for cc in range(_HD // 16):
                        sl = pl.ds(cc * 16, 16)
                        g1_v[k][r, sl] = (g1_v[k][r, sl] * g2_v[k][r, sl]
                                          * g3_v[k][r, sl])
                    return carry2
                lax.fori_loop(0, _K2, mrow, 0)
                pltpu.sync_copy(g1_v[k], acc_sh.at[dv_v[k]], add=True)

        issue_idx(0, 0)
        drain_idx(0, 0)
        issue_gather(0, 0)
        issue_idx(1, 1)

        def step(q, carry):
            for p in range(2):
                t = 2 * q + p
                k = p
                drain_idx(t + 1, 1 - k)
                issue_gather(t + 1, 1 - k)
                drain_gather(t, k)
                process(t, k)
                issue_idx(t + 2, k)
            return carry
        lax.fori_loop(0, (niter + 1) // 2, step, 0)

    def edge_stream(ms_h):
        nchunks = E // _K2

        def cid(t):
            return t * 16 + s

        def issue(t, k):
            @pl.when(cid(t) < nchunks)
            def _():
                off = cid(t) * _K2
                pltpu.async_copy(ie_h.at[pl.ds(off, _K2)], dv_v[k], sidx[k])
                pltpu.async_copy(ms_h.at[pl.ds(off, _K2)], g3_v[k], sidx[k])

        def drain(t, k):
            @pl.when(cid(t) < nchunks)
            def _():
                off = cid(t) * _K2
                pltpu.make_async_copy(ie_h.at[pl.ds(off, _K2)], dv_v[k], sidx[k]).wait()
                pltpu.make_async_copy(ms_h.at[pl.ds(off, _K2)], g3_v[k], sidx[k]).wait()

        def scat(t, k):
            @pl.when(cid(t) < nchunks)
            def _():
                pltpu.sync_copy(g3_v[k], acc_sh.at[dv_v[k]], add=True)

        issue(0, 0)
        issue(1, 1)

        def estep(q, carry):
            for p in range(2):
                t = 2 * q + p
                k = p
                drain(t, k)
                scat(t, k)
                issue(t + 2, k)
            return carry
        niter = (nchunks + 15) // 16
        lax.fori_loop(0, (niter + 1) // 2, estep, 0)

    @pl.when(c == 0)
    def _half0():
        triplet_stream(nb0_h, ro0_h, ikj_h, iji_h, d2_h, sh20_h, T2)
        triplet_stream(nb0_h, ro0_h, ijj_h, ijp_h, d1_h, sh10_h, T1)
        edge_stream(ms0_h)

    @pl.when(c == 1)
    def _half1():
        triplet_stream(nb1_h, ro1_h, ikj_h, iji_h, d2_h, sh21_h, T2)
        triplet_stream(nb1_h, ro1_h, ijj_h, ijp_h, d1_h, sh11_h, T1)
        edge_stream(ms1_h)

    plsc.subcore_barrier()
    pltpu.sync_copy(acc_sh.at[pl.ds(s * _RPS, _RPS)],
                    out_h.at[c, pl.ds(s * _RPS, _RPS)])


def _sc2(nb0, nb1, ro0, ro1, ms0, ms1, sh20, sh21, sh10, sh11,
         idx_kj, idx_ji, idx_jj_pair, idx_ji_pair, i_idx):
    dst2 = jnp.take(i_idx, idx_ji, axis=0)
    dst1 = jnp.take(i_idx, idx_ji_pair, axis=0)
    mesh = plsc.VectorSubcoreMesh(core_axis_name="c", subcore_axis_name="s")
    f = functools.partial(
        pl.kernel,
        out_type=jax.ShapeDtypeStruct((2, _NP, _HD), jnp.float32),
        mesh=mesh,
        scratch_types=[
            [pltpu.VMEM((_K2, _HD), jnp.float32),
             pltpu.VMEM((_K2, _HD), jnp.float32)],
            [pltpu.VMEM((_K2, _HD), jnp.float32),
             pltpu.VMEM((_K2, _HD), jnp.float32)],
            [pltpu.VMEM((_K2, _HD), jnp.float32),
             pltpu.VMEM((_K2, _HD), jnp.float32)],
            [pltpu.VMEM((_K2,), jnp.int32), pltpu.VMEM((_K2,), jnp.int32)],
            [pltpu.VMEM((_K2,), jnp.int32), pltpu.VMEM((_K2,), jnp.int32)],
            [pltpu.VMEM((_K2,), jnp.int32), pltpu.VMEM((_K2,), jnp.int32)],
            pltpu.VMEM_SHARED((_NP, _HD), jnp.float32),
            [pltpu.SemaphoreType.DMA, pltpu.SemaphoreType.DMA],
            [pltpu.SemaphoreType.DMA, pltpu.SemaphoreType.DMA],
        ],
    )(_sc2_body)
    p = f(nb0, ro0, ms0, sh20, sh10, nb1, ro1, ms1, sh21, sh11,
          idx_kj, idx_ji, dst2, idx_jj_pair, idx_ji_pair, dst1, i_idx)
    return p[0, :N], p[1, :N]


def kernel(x, rbf, sbf2, sbf1, idx_kj, idx_ji, idx_jj_pair, idx_ji_pair,
           edge_index, params):
    p = params
    j_idx = edge_index[0]
    i_idx = edge_index[1]

    wti = jnp.concatenate([p['Wji'][:D], p['Wkj'][:D]], axis=1)
    wtj = jnp.concatenate([p['Wji'][D:2 * D], p['Wkj'][D:2 * D]], axis=1)
    wr4 = jnp.concatenate(
        [p['Wji'][2 * D:], p['Wkj'][2 * D:], p['Wrbf'], p['Wrbfo']], axis=1)
    wsc = jnp.concatenate(
        [p['Wout'], p['Watt'], jnp.zeros((D, D // 2 - 2), jnp.float32)], axis=1)
    bsc = jnp.concatenate(
        [p['bout'], jnp.zeros((D // 2 - 1,), jnp.float32)])[None]
    row_b = lambda b: b[None]

    h, ti, tj = _tc1(x, p['Wx1'], row_b(p['bx1']), wti, wtj)

    sh20, sh21 = _sbf_mlp(sbf2, p['Wsbf1'], row_b(p['bsbf1']),
                          p['Wsbf2'], row_b(p['bsbf2']))
    sh10, sh11 = _sbf_mlp(sbf1, p['Wsbf1'], row_b(p['bsbf1']),
                          p['Wsbf2'], row_b(p['bsbf2']))

    ga, gb = _sc1(ti, tj, edge_index)

    nb0, nb1, ro0, ro1, ms0, ms1 = _tc24(
        rbf, ga, gb, wr4, row_b(p['bji']), row_b(p['bkj']))

    a0, a1 = _sc2(nb0, nb1, ro0, ro1, ms0, ms1,
                  sh20, sh21, sh10, sh11,
                  idx_kj, idx_ji, idx_jj_pair, idx_ji_pair, i_idx)

    tc5_weights = [
        p['Wx2'], row_b(p['bx2']),
        p['Wr1a'], row_b(p['br1a']), p['Wr1b'], row_b(p['br1b']),
        p['Wr2a'], row_b(p['br2a']), p['Wr2b'], row_b(p['br2b']),
        p['Wr3a'], row_b(p['br3a']), p['Wr3b'], row_b(p['br3b']),
        p['Wo1'], row_b(p['bo1']), p['Wo2'], row_b(p['bo2']),
        p['Wo3'], row_b(p['bo3']),
        wsc, bsc,
    ]
    h_out, s = _tc5(x, h, a0, a1, tc5_weights)

    out_final = s[:, 0:1][None]
    att_score = s[:, 1:2][None]
    return (h_out, out_final, att_score)
